# Initial kernel scaffold; baseline (speedup 1.0000x reference)
import jax
import jax.numpy as jnp
from jax.experimental import pallas as pl


def kernel(atom_types, frac_coords, lattices, edge_index, node2graph, t, params):
    raise NotImplementedError("write your pallas kernel here")


# speedup vs baseline: 10.2526x; 10.2526x over previous
import dataclasses
import functools
import math

import jax
import jax.numpy as jnp
import numpy as np
from jax import lax
from jax.experimental import pallas as pl
from jax.experimental.pallas import tpu as pltpu
from jax.experimental.pallas import tpu_sc as plsc

NFREQ = 10


_SC_CORES = 2
_SC_TILES = 16
_SC_W = 128


def _sc_mesh():
    return plsc.VectorSubcoreMesh(
        core_axis_name="c", subcore_axis_name="s",
        num_cores=_SC_CORES, num_subcores=_SC_TILES)


def _sc_gather_pairs(t1, i1, t2, i2):
    n, d = t1.shape
    e = i1.shape[1]
    w = _SC_W

    @functools.partial(
        pl.kernel,
        out_type=(jax.ShapeDtypeStruct((e, d), jnp.float32),
                  jax.ShapeDtypeStruct((e, d), jnp.float32)),
        mesh=_sc_mesh())
    def k(t1_hbm, i1_hbm, t2_hbm, i2_hbm, o1_hbm, o2_hbm):
        def body(i1_v, i2_v, o1_v, o2_v):
            pltpu.sync_copy(t1_hbm.at[i1_v.at[0]], o1_v)
            pltpu.sync_copy(t2_hbm.at[i2_v.at[0]], o2_v)

        pltpu.emit_pipeline(
            body,
            grid=(e // w,),
            in_specs=[pl.BlockSpec((1, w), lambda i: (0, i)),
                      pl.BlockSpec((1, w), lambda i: (0, i))],
            out_specs=[pl.BlockSpec((w, d), lambda i: (i, 0)),
                       pl.BlockSpec((w, d), lambda i: (i, 0))],
            core_axis_name=("c", "s"),
            dimension_semantics=(pltpu.PARALLEL,),
        )(i1_hbm, i2_hbm, o1_hbm, o2_hbm)

    return k(t1, i1, t2, i2)


def _sc_fdiff(frac_flat, src, dst):
    e = src.shape[1]
    w = _SC_W
    nflat = frac_flat.shape[0]

    cp = pltpu.CompilerParams()
    if "needs_layout_passes" in pltpu.CompilerParams.__dataclass_fields__:
        cp = dataclasses.replace(cp, needs_layout_passes=False)

    @functools.partial(
        pl.kernel,
        out_type=jax.ShapeDtypeStruct((4, e), jnp.float32),
        mesh=_sc_mesh(),
        compiler_params=cp,
        scratch_types=[pltpu.VMEM((nflat,), jnp.float32)])
    def k(f_hbm, s_hbm, d_hbm, o_hbm, tbl):
        pltpu.sync_copy(f_hbm, tbl)

        def body(s_v, d_v, o_v):
            for gi in range(w // 16):
                sl = pl.ds(gi * 16, 16)
                s16 = s_v[0, sl] * 4
                d16 = d_v[0, sl] * 4
                for c in range(3):
                    fs = plsc.load_gather(tbl, [s16 + c])
                    fd = plsc.load_gather(tbl, [d16 + c])
                    o_v[c, sl] = fd - fs
                o_v[3, sl] = jnp.zeros((16,), jnp.float32)

        pltpu.emit_pipeline(
            body,
            grid=(e // w,),
            in_specs=[pl.BlockSpec((1, w), lambda i: (0, i)),
                      pl.BlockSpec((1, w), lambda i: (0, i))],
            out_specs=[pl.BlockSpec((4, w), lambda i: (0, i))],
            core_axis_name=("c", "s"),
            dimension_semantics=(pltpu.PARALLEL,),
        )(s_hbm, d_hbm, o_hbm)

    return k(frac_flat, src, dst)


def _sc_scatter_rows(vals, idx, zeros):
    e, d = vals.shape
    n = zeros.shape[0]
    w = _SC_W
    rows = n // _SC_TILES

    @functools.partial(
        pl.kernel,
        out_type=jax.ShapeDtypeStruct((_SC_CORES, n, d), jnp.float32),
        mesh=_sc_mesh(),
        scratch_types=[pltpu.VMEM_SHARED((n, d), jnp.float32)])
    def k(v_hbm, i_hbm, z_hbm, o_hbm, acc):
        cid = lax.axis_index("c")
        sid = lax.axis_index("s")
        pltpu.sync_copy(z_hbm.at[pl.ds(sid * rows, rows)],
                        acc.at[pl.ds(sid * rows, rows)])
        plsc.subcore_barrier()

        def body(v_v, i_v):
            pltpu.sync_copy(v_v, acc.at[i_v.at[0]], add=True)

        pltpu.emit_pipeline(
            body,
            grid=(e // w,),
            in_specs=[pl.BlockSpec((w, d), lambda i: (i, 0)),
                      pl.BlockSpec((1, w), lambda i: (0, i))],
            out_specs=[],
            core_axis_name=("c", "s"),
            dimension_semantics=(pltpu.PARALLEL,),
        )(v_hbm, i_hbm)

        plsc.subcore_barrier()
        pltpu.sync_copy(acc.at[pl.ds(sid * rows, rows)],
                        o_hbm.at[cid].at[pl.ds(sid * rows, rows)])

    return k(vals, idx, zeros)


def _sc_counts(idx, ones, zeros):
    e = idx.shape[1]
    n, dc = zeros.shape
    w = _SC_W
    rows = n // _SC_TILES

    @functools.partial(
        pl.kernel,
        out_type=jax.ShapeDtypeStruct((_SC_CORES, n, dc), jnp.float32),
        mesh=_sc_mesh(),
        scratch_types=[pltpu.VMEM((w, dc), jnp.float32),
                       pltpu.VMEM_SHARED((n, dc), jnp.float32)])
    def k(i_hbm, one_hbm, z_hbm, o_hbm, ones_v, acc):
        cid = lax.axis_index("c")
        sid = lax.axis_index("s")
        pltpu.sync_copy(one_hbm, ones_v)
        pltpu.sync_copy(z_hbm.at[pl.ds(sid * rows, rows)],
                        acc.at[pl.ds(sid * rows, rows)])
        plsc.subcore_barrier()

        def body(i_v):
            pltpu.sync_copy(ones_v, acc.at[i_v.at[0]], add=True)

        pltpu.emit_pipeline(
            body,
            grid=(e // w,),
            in_specs=[pl.BlockSpec((1, w), lambda i: (0, i))],
            out_specs=[],
            core_axis_name=("c", "s"),
            dimension_semantics=(pltpu.PARALLEL,),
        )(i_hbm)

        plsc.subcore_barrier()
        pltpu.sync_copy(acc.at[pl.ds(sid * rows, rows)],
                        o_hbm.at[cid].at[pl.ds(sid * rows, rows)])

    return k(idx, ones, zeros)


_BN = 1000
_BE = 4000


def _silu(x):
    return x * jax.nn.sigmoid(x)


def _onehot_from(ids, nclass):
    return (ids[:, None] == lax.broadcasted_iota(jnp.int32, (ids.shape[0], nclass), 1)
            ).astype(jnp.float32)


def _tc_tables(lat9, wlat_all, emb_pad, wtop, t, wbot):
    g = lat9.shape[0]

    def body(l_ref, wl_ref, e_ref, wt_ref, t_ref, wb_ref,
             latt_ref, t1_ref, t2_ref):
        l = l_ref[...]
        cols = []
        for i in range(3):
            for kk in range(3):
                acc = l[:, 3 * i + 0] * l[:, 3 * kk + 0]
                acc = acc + l[:, 3 * i + 1] * l[:, 3 * kk + 1]
                acc = acc + l[:, 3 * i + 2] * l[:, 3 * kk + 2]
                cols.append(acc)
        for _ in range(7):
            cols.append(jnp.zeros((g,), jnp.float32))
        lat_ip = jnp.stack(cols, axis=1)
        for layer in range(4):
            wl = wl_ref[pl.ds(16 * layer, 16), :]
            latt_ref[pl.ds(g * layer, g), :] = jnp.dot(
                lat_ip, wl, preferred_element_type=jnp.float32)
        t1_ref[...] = jnp.dot(e_ref[...], wt_ref[...],
                              preferred_element_type=jnp.float32)
        t2_ref[...] = jnp.dot(t_ref[...], wb_ref[...],
                              preferred_element_type=jnp.float32)

    return pl.pallas_call(
        body,
        out_shape=(jax.ShapeDtypeStruct((4 * g, 128), jnp.float32),
                   jax.ShapeDtypeStruct((128, 128), jnp.float32),
                   jax.ShapeDtypeStruct((g, 128), jnp.float32)),
    )(lat9, wlat_all, emb_pad, wtop, t, wbot)


def _tc_h0(at3, n2g3, t1, t2, b, latt0, wi, wj, n):
    nb = n // _BN

    def body(at_ref, ng_ref, t1_ref, t2_ref, b_ref, lt_ref, wi_ref, wj_ref,
             h_ref, hs_ref, hd_ref):
        oh_at = _onehot_from(at_ref[0, 0, :], 128)
        oh_g = _onehot_from(ng_ref[0, 0, :], 64)
        h = (jnp.dot(oh_at, t1_ref[...], preferred_element_type=jnp.float32)
             + jnp.dot(oh_g, t2_ref[...], preferred_element_type=jnp.float32)
             + b_ref[...])
        h_ref[...] = h
        hs_ref[...] = (jnp.dot(h, wi_ref[...], preferred_element_type=jnp.float32)
                       + jnp.dot(oh_g, lt_ref[...], preferred_element_type=jnp.float32))
        hd_ref[...] = jnp.dot(h, wj_ref[...], preferred_element_type=jnp.float32)

    fixed = pl.BlockSpec(None, None)
    return pl.pallas_call(
        body,
        grid=(nb,),
        in_specs=[pl.BlockSpec((1, 1, _BN), lambda i: (i, 0, 0)),
                  pl.BlockSpec((1, 1, _BN), lambda i: (i, 0, 0)),
                  fixed, fixed, fixed, fixed, fixed, fixed],
        out_specs=[pl.BlockSpec((_BN, 128), lambda i: (i, 0)),
                   pl.BlockSpec((_BN, 128), lambda i: (i, 0)),
                   pl.BlockSpec((_BN, 128), lambda i: (i, 0))],
        out_shape=(jax.ShapeDtypeStruct((n, 128), jnp.float32),
                   jax.ShapeDtypeStruct((n, 128), jnp.float32),
                   jax.ShapeDtypeStruct((n, 128), jnp.float32)),
    )(at3, n2g3, t1, t2, b, latt0, wi, wj)


def _tc_dis(fdT, fmapT):
    e = fdT.shape[1]
    be = 2560
    nb = e // be

    def body(d_ref, f_ref, o_ref):
        ang_t = jnp.dot(f_ref[...], d_ref[...],
                        preferred_element_type=jnp.float32)
        row = lax.broadcasted_iota(jnp.int32, ang_t.shape, 0)
        dis_t = jnp.where(row < 32, jnp.sin(ang_t), jnp.cos(ang_t))
        o_ref[...] = dis_t.T

    fixed = pl.BlockSpec(None, None)
    return pl.pallas_call(
        body,
        grid=(nb,),
        in_specs=[pl.BlockSpec((4, be), lambda i: (0, i)),
                  fixed],
        out_specs=pl.BlockSpec((be, 64), lambda i: (i, 0)),
        out_shape=jax.ShapeDtypeStruct((e, 64), jnp.float32),
    )(fdT, fmapT)


def _tc_edge(ghs, ghd, dis, wdis, w2, b1, b2):
    e = ghs.shape[0]
    nb = e // _BE

    def body(s_ref, d_ref, x_ref, wd_ref, w2_ref, b1_ref, b2_ref, o_ref):
        pre = (s_ref[...] + d_ref[...] + b1_ref[...]
               + jnp.dot(x_ref[...], wd_ref[...],
                         preferred_element_type=jnp.float32))
        s1 = _silu(pre)
        z = jnp.dot(s1, w2_ref[...], preferred_element_type=jnp.float32) + b2_ref[...]
        o_ref[...] = _silu(z)

    fixed = pl.BlockSpec(None, None)
    return pl.pallas_call(
        body,
        grid=(nb,),
        in_specs=[pl.BlockSpec((_BE, 128), lambda i: (i, 0)),
                  pl.BlockSpec((_BE, 128), lambda i: (i, 0)),
                  pl.BlockSpec((_BE, 64), lambda i: (i, 0)),
                  fixed, fixed, fixed, fixed],
        out_specs=pl.BlockSpec((_BE, 128), lambda i: (i, 0)),
        out_shape=jax.ShapeDtypeStruct((e, 128), jnp.float32),
    )(ghs, ghd, dis, wdis, w2, b1, b2)


def _tc_node(h, agg2, cnt2, w1h, w1a, b1, w2, b2, nxt):
    n = h.shape[0]
    nb = n // _BN

    def body(h_ref, a_ref, c_ref, w1h_ref, w1a_ref, b1_ref, w2_ref, b2_ref,
             *rest):
        h = h_ref[...]
        cnt = c_ref[0] + c_ref[1]
        denom = jnp.maximum(cnt[:, 0:1], 1.0)
        agg = (a_ref[0] + a_ref[1]) / denom
        z1 = _silu(jnp.dot(h, w1h_ref[...], preferred_element_type=jnp.float32)
                   + jnp.dot(agg, w1a_ref[...], preferred_element_type=jnp.float32)
                   + b1_ref[...])
        z2 = _silu(jnp.dot(z1, w2_ref[...], preferred_element_type=jnp.float32)
                   + b2_ref[...])
        hn = h + z2
        if nxt is None:
            (ho_ref,) = rest
            ho_ref[...] = hn
        else:
            ng_ref, lt_ref, wi_ref, wj_ref, ho_ref, hs_ref, hd_ref = rest
            ho_ref[...] = hn
            oh_g = _onehot_from(ng_ref[0, 0, :], 64)
            hs_ref[...] = (jnp.dot(hn, wi_ref[...], preferred_element_type=jnp.float32)
                           + jnp.dot(oh_g, lt_ref[...], preferred_element_type=jnp.float32))
            hd_ref[...] = jnp.dot(hn, wj_ref[...], preferred_element_type=jnp.float32)

    fixed = pl.BlockSpec(None, None)
    in_specs = [pl.BlockSpec((_BN, 128), lambda i: (i, 0)),
                pl.BlockSpec((2, _BN, 128), lambda i: (0, i, 0)),
                pl.BlockSpec((2, _BN, 128), lambda i: (0, i, 0)),
                fixed, fixed, fixed, fixed, fixed]
    args = [h, agg2, cnt2, w1h, w1a, b1, w2, b2]
    out_specs = [pl.BlockSpec((_BN, 128), lambda i: (i, 0))]
    out_shape = [jax.ShapeDtypeStruct((n, 128), jnp.float32)]
    if nxt is not None:
        n2g3, latt_l, wi, wj = nxt
        in_specs += [pl.BlockSpec((1, 1, _BN), lambda i: (i, 0, 0)),
                     fixed, fixed, fixed]
        args += [n2g3, latt_l, wi, wj]
        out_specs += [pl.BlockSpec((_BN, 128), lambda i: (i, 0)),
                      pl.BlockSpec((_BN, 128), lambda i: (i, 0))]
        out_shape += [jax.ShapeDtypeStruct((n, 128), jnp.float32),
                      jax.ShapeDtypeStruct((n, 128), jnp.float32)]
    res = pl.pallas_call(
        body,
        grid=(nb,),
        in_specs=in_specs,
        out_specs=out_specs,
        out_shape=out_shape,
    )(*args)
    return res if nxt is not None else (res[0],)


def _tc_pool(h, n2g3, coordw, n):
    nb = n // _BN

    def body(h_ref, ng_ref, cw_ref, co_ref, gs_ref, gc_ref):
        i = pl.program_id(0)
        h = h_ref[...]
        co_ref[...] = jnp.dot(h, cw_ref[...], preferred_element_type=jnp.float32)
        ids = ng_ref[0, 0, :]
        oht = (lax.broadcasted_iota(jnp.int32, (64, _BN), 0) == ids[None, :]
               ).astype(jnp.float32)

        @pl.when(i == 0)
        def _():
            gs_ref[...] = jnp.zeros_like(gs_ref)
            gc_ref[...] = jnp.zeros_like(gc_ref)

        gs_ref[...] += jnp.dot(oht, h, preferred_element_type=jnp.float32)
        gc_ref[...] += jnp.dot(oht, jnp.ones((_BN, 128), jnp.float32),
                               preferred_element_type=jnp.float32)

    fixed = pl.BlockSpec(None, None)
    return pl.pallas_call(
        body,
        grid=(nb,),
        in_specs=[pl.BlockSpec((_BN, 128), lambda i: (i, 0)),
                  pl.BlockSpec((1, 1, _BN), lambda i: (i, 0, 0)),
                  fixed],
        out_specs=[pl.BlockSpec((_BN, 8), lambda i: (i, 0)),
                   pl.BlockSpec((64, 128), lambda i: (0, 0)),
                   pl.BlockSpec((64, 128), lambda i: (0, 0))],
        out_shape=(jax.ShapeDtypeStruct((n, 8), jnp.float32),
                   jax.ShapeDtypeStruct((64, 128), jnp.float32),
                   jax.ShapeDtypeStruct((64, 128), jnp.float32)),
    )(h, n2g3, coordw)


def _tc_lattice(gsum, gcnt, latw, lat9):
    g = lat9.shape[0]

    def body(gs_ref, gc_ref, w_ref, l_ref, o_ref):
        gf = gs_ref[...] / jnp.maximum(gc_ref[...], 1.0)
        gl = jnp.dot(gf, w_ref[...], preferred_element_type=jnp.float32)
        l = l_ref[...]
        cols = []
        for i in range(3):
            for kk in range(3):
                acc = gl[:, 3 * i + 0] * l[:, 0 + kk]
                acc = acc + gl[:, 3 * i + 1] * l[:, 3 + kk]
                acc = acc + gl[:, 3 * i + 2] * l[:, 6 + kk]
                cols.append(acc)
        for _ in range(7):
            cols.append(jnp.zeros((g,), jnp.float32))
        o_ref[...] = jnp.stack(cols, axis=1)

    return pl.pallas_call(
        body,
        out_shape=jax.ShapeDtypeStruct((g, 16), jnp.float32),
    )(gsum, gcnt, latw, lat9)


def kernel(atom_types, frac_coords, lattices, edge_index, node2graph, t, params):
    n = atom_types.shape[0]
    e = edge_index.shape[1]
    g = lattices.shape[0]
    hid = 128

    src = edge_index[0].astype(jnp.int32).reshape(1, e)
    dst = edge_index[1].astype(jnp.int32).reshape(1, e)
    at3 = atom_types.astype(jnp.int32).reshape(n // _BN, 1, _BN)
    n2g3 = node2graph.astype(jnp.int32).reshape(n // _BN, 1, _BN)

    p = params
    layers = p["layers"]

    wtop = p["atom_latent_W"][:hid]
    wbot = p["atom_latent_W"][hid:]
    b_al = p["atom_latent_b"].reshape(1, hid)
    emb_pad = jnp.zeros((128, hid), jnp.float32).at[:p["node_emb"].shape[0]].set(
        p["node_emb"])
    lat9 = jnp.pad(lattices.reshape(g, 9), ((0, 0), (0, 7)))
    wlat_all = jnp.concatenate(
        [jnp.pad(lp["eW1"][2 * hid:2 * hid + 9], ((0, 7), (0, 0)))
         for lp in layers], axis=0)
    wi = [lp["eW1"][:hid] for lp in layers]
    wj = [lp["eW1"][hid:2 * hid] for lp in layers]
    wdis = []
    for lp in layers:
        wd = lp["eW1"][2 * hid + 9:]
        wdis.append(jnp.concatenate([
            wd[:30], jnp.zeros((2, hid), jnp.float32),
            wd[30:], jnp.zeros((2, hid), jnp.float32)], axis=0))
    eb1 = [lp["eb1"].reshape(1, hid) for lp in layers]
    ew2 = [lp["eW2"] for lp in layers]
    eb2 = [lp["eb2"].reshape(1, hid) for lp in layers]
    nw1h = [lp["nW1"][:hid] for lp in layers]
    nw1a = [lp["nW1"][hid:] for lp in layers]
    nb1 = [lp["nb1"].reshape(1, hid) for lp in layers]
    nw2 = [lp["nW2"] for lp in layers]
    nb2 = [lp["nb2"].reshape(1, hid) for lp in layers]
    coordw = jnp.pad(p["coord_W"], ((0, 0), (0, 5)))
    latw = jnp.pad(p["lattice_W"], ((0, 0), (0, 7)))

    freqs = 2.0 * math.pi * np.arange(NFREQ, dtype=np.float32)
    fmap_np = np.zeros((64, 4), np.float32)
    for j in range(3):
        for f in range(NFREQ):
            fmap_np[j * NFREQ + f, j] = freqs[f]
            fmap_np[32 + j * NFREQ + f, j] = freqs[f]
    fmap_t = jnp.asarray(fmap_np)

    frac_flat = jnp.pad(frac_coords, ((0, 0), (0, 1))).reshape(-1)
    npad = ((n + 8 * _SC_TILES - 1) // (8 * _SC_TILES)) * (8 * _SC_TILES)
    zeros_nd = jnp.zeros((npad, hid), jnp.float32)
    zeros_nc = jnp.zeros((npad, hid), jnp.float32)
    ones_w = jnp.ones((_SC_W, hid), jnp.float32)

    latt_all, t1, t2 = _tc_tables(lat9, wlat_all, emb_pad, wtop, t, wbot)
    cnt2 = _sc_counts(src, ones_w, zeros_nc)
    fdT = _sc_fdiff(frac_flat, src, dst)
    dis = _tc_dis(fdT, fmap_t)
    h, hs, hd = _tc_h0(at3, n2g3, t1, t2, b_al,
                       lax.slice_in_dim(latt_all, 0, g), wi[0], wj[0], n)

    for l in range(4):
        ghs, ghd = _sc_gather_pairs(hs, src, hd, dst)
        ef = _tc_edge(ghs, ghd, dis, wdis[l], ew2[l], eb1[l], eb2[l])
        agg2 = _sc_scatter_rows(ef, src, zeros_nd)
        if l < 3:
            nxt = (n2g3, lax.slice_in_dim(latt_all, (l + 1) * g, (l + 2) * g),
                   wi[l + 1], wj[l + 1])
            h, hs, hd = _tc_node(h, agg2, cnt2, nw1h[l], nw1a[l], nb1[l],
                                 nw2[l], nb2[l], nxt)
        else:
            (h,) = _tc_node(h, agg2, cnt2, nw1h[l], nw1a[l], nb1[l],
                            nw2[l], nb2[l], None)

    coord8, gsum, gcnt = _tc_pool(h, n2g3, coordw, n)
    lo16 = _tc_lattice(gsum, gcnt, latw, lat9)

    coord_out = coord8[:, :3]
    lattice_out = lo16[:, :9].reshape(g, 3, 3)
    return lattice_out, coord_out


# speedup vs baseline: 10.2712x; 1.0018x over previous
import dataclasses
import functools
import math

import jax
import jax.numpy as jnp
import numpy as np
from jax import lax
from jax.experimental import pallas as pl
from jax.experimental.pallas import tpu as pltpu
from jax.experimental.pallas import tpu_sc as plsc

NFREQ = 10


_SC_CORES = 2
_SC_TILES = 16
_SC_W = 128


def _sc_mesh():
    return plsc.VectorSubcoreMesh(
        core_axis_name="c", subcore_axis_name="s",
        num_cores=_SC_CORES, num_subcores=_SC_TILES)


def _sc_gather_pairs(t1, i1, t2, i2):
    n, d = t1.shape
    e = i1.shape[1]
    w = _SC_W

    @functools.partial(
        pl.kernel,
        out_type=(jax.ShapeDtypeStruct((e, d), jnp.float32),
                  jax.ShapeDtypeStruct((e, d), jnp.float32)),
        mesh=_sc_mesh())
    def k(t1_hbm, i1_hbm, t2_hbm, i2_hbm, o1_hbm, o2_hbm):
        def body(i1_v, i2_v, o1_v, o2_v):
            pltpu.sync_copy(t1_hbm.at[i1_v.at[0]], o1_v)
            pltpu.sync_copy(t2_hbm.at[i2_v.at[0]], o2_v)

        pltpu.emit_pipeline(
            body,
            grid=(e // w,),
            in_specs=[pl.BlockSpec((1, w), lambda i: (0, i)),
                      pl.BlockSpec((1, w), lambda i: (0, i))],
            out_specs=[pl.BlockSpec((w, d), lambda i: (i, 0)),
                       pl.BlockSpec((w, d), lambda i: (i, 0))],
            core_axis_name=("c", "s"),
            dimension_semantics=(pltpu.PARALLEL,),
        )(i1_hbm, i2_hbm, o1_hbm, o2_hbm)

    return k(t1, i1, t2, i2)


def _sc_fdiff(frac_flat, src, dst):
    e = src.shape[1]
    w = _SC_W
    nflat = frac_flat.shape[0]

    cp = pltpu.CompilerParams()
    if "needs_layout_passes" in pltpu.CompilerParams.__dataclass_fields__:
        cp = dataclasses.replace(cp, needs_layout_passes=False)

    @functools.partial(
        pl.kernel,
        out_type=jax.ShapeDtypeStruct((4, e), jnp.float32),
        mesh=_sc_mesh(),
        compiler_params=cp,
        scratch_types=[pltpu.VMEM((nflat,), jnp.float32)])
    def k(f_hbm, s_hbm, d_hbm, o_hbm, tbl):
        pltpu.sync_copy(f_hbm, tbl)

        def body(s_v, d_v, o_v):
            for gi in range(w // 16):
                sl = pl.ds(gi * 16, 16)
                s16 = s_v[0, sl] * 4
                d16 = d_v[0, sl] * 4
                for c in range(3):
                    fs = plsc.load_gather(tbl, [s16 + c])
                    fd = plsc.load_gather(tbl, [d16 + c])
                    o_v[c, sl] = fd - fs
                o_v[3, sl] = jnp.zeros((16,), jnp.float32)

        pltpu.emit_pipeline(
            body,
            grid=(e // w,),
            in_specs=[pl.BlockSpec((1, w), lambda i: (0, i)),
                      pl.BlockSpec((1, w), lambda i: (0, i))],
            out_specs=[pl.BlockSpec((4, w), lambda i: (0, i))],
            core_axis_name=("c", "s"),
            dimension_semantics=(pltpu.PARALLEL,),
        )(s_hbm, d_hbm, o_hbm)

    return k(frac_flat, src, dst)


def _sc_scatter_rows(vals, idx, zeros):
    e, d = vals.shape
    n = zeros.shape[0]
    w = _SC_W
    rows = n // _SC_TILES

    @functools.partial(
        pl.kernel,
        out_type=jax.ShapeDtypeStruct((_SC_CORES, n, d), jnp.float32),
        mesh=_sc_mesh(),
        scratch_types=[pltpu.VMEM_SHARED((n, d), jnp.float32)])
    def k(v_hbm, i_hbm, z_hbm, o_hbm, acc):
        cid = lax.axis_index("c")
        sid = lax.axis_index("s")
        pltpu.sync_copy(z_hbm.at[pl.ds(sid * rows, rows)],
                        acc.at[pl.ds(sid * rows, rows)])
        plsc.subcore_barrier()

        def body(v_v, i_v):
            pltpu.sync_copy(v_v, acc.at[i_v.at[0]], add=True)

        pltpu.emit_pipeline(
            body,
            grid=(e // w,),
            in_specs=[pl.BlockSpec((w, d), lambda i: (i, 0)),
                      pl.BlockSpec((1, w), lambda i: (0, i))],
            out_specs=[],
            core_axis_name=("c", "s"),
            dimension_semantics=(pltpu.PARALLEL,),
        )(v_hbm, i_hbm)

        plsc.subcore_barrier()
        pltpu.sync_copy(acc.at[pl.ds(sid * rows, rows)],
                        o_hbm.at[cid].at[pl.ds(sid * rows, rows)])

    return k(vals, idx, zeros)


def _sc_counts(idx, ones, zeros):
    e = idx.shape[1]
    n, dc = zeros.shape
    w = _SC_W
    rows = n // _SC_TILES

    @functools.partial(
        pl.kernel,
        out_type=jax.ShapeDtypeStruct((_SC_CORES, n, dc), jnp.float32),
        mesh=_sc_mesh(),
        scratch_types=[pltpu.VMEM((w, dc), jnp.float32),
                       pltpu.VMEM_SHARED((n, dc), jnp.float32)])
    def k(i_hbm, one_hbm, z_hbm, o_hbm, ones_v, acc):
        cid = lax.axis_index("c")
        sid = lax.axis_index("s")
        pltpu.sync_copy(one_hbm, ones_v)
        pltpu.sync_copy(z_hbm.at[pl.ds(sid * rows, rows)],
                        acc.at[pl.ds(sid * rows, rows)])
        plsc.subcore_barrier()

        def body(i_v):
            pltpu.sync_copy(ones_v, acc.at[i_v.at[0]], add=True)

        pltpu.emit_pipeline(
            body,
            grid=(e // w,),
            in_specs=[pl.BlockSpec((1, w), lambda i: (0, i))],
            out_specs=[],
            core_axis_name=("c", "s"),
            dimension_semantics=(pltpu.PARALLEL,),
        )(i_hbm)

        plsc.subcore_barrier()
        pltpu.sync_copy(acc.at[pl.ds(sid * rows, rows)],
                        o_hbm.at[cid].at[pl.ds(sid * rows, rows)])

    return k(idx, ones, zeros)


_BN = 1000
_BE = 4000


def _silu(x):
    return x * jax.nn.sigmoid(x)


def _onehot_from(ids, nclass):
    return (ids[:, None] == lax.broadcasted_iota(jnp.int32, (ids.shape[0], nclass), 1)
            ).astype(jnp.float32)


def _tc_tables(lat9, wlat_all, emb_pad, wtop, t, wbot):
    g = lat9.shape[0]

    def body(l_ref, wl_ref, e_ref, wt_ref, t_ref, wb_ref,
             latt_ref, t1_ref, t2_ref):
        l = l_ref[...]
        cols = []
        for i in range(3):
            for kk in range(3):
                acc = l[:, 3 * i + 0] * l[:, 3 * kk + 0]
                acc = acc + l[:, 3 * i + 1] * l[:, 3 * kk + 1]
                acc = acc + l[:, 3 * i + 2] * l[:, 3 * kk + 2]
                cols.append(acc)
        for _ in range(7):
            cols.append(jnp.zeros((g,), jnp.float32))
        lat_ip = jnp.stack(cols, axis=1)
        for layer in range(4):
            wl = wl_ref[pl.ds(16 * layer, 16), :]
            latt_ref[pl.ds(g * layer, g), :] = jnp.dot(
                lat_ip, wl, preferred_element_type=jnp.float32)
        t1_ref[...] = jnp.dot(e_ref[...], wt_ref[...],
                              preferred_element_type=jnp.float32)
        t2_ref[...] = jnp.dot(t_ref[...], wb_ref[...],
                              preferred_element_type=jnp.float32)

    return pl.pallas_call(
        body,
        out_shape=(jax.ShapeDtypeStruct((4 * g, 128), jnp.float32),
                   jax.ShapeDtypeStruct((128, 128), jnp.float32),
                   jax.ShapeDtypeStruct((g, 128), jnp.float32)),
    )(lat9, wlat_all, emb_pad, wtop, t, wbot)


def _tc_h0(at3, n2g3, t1, t2, b, latt0, wi, wj, n):
    nb = n // _BN

    def body(at_ref, ng_ref, t1_ref, t2_ref, b_ref, lt_ref, wi_ref, wj_ref,
             h_ref, hs_ref, hd_ref):
        oh_at = _onehot_from(at_ref[0, 0, :], 128)
        oh_g = _onehot_from(ng_ref[0, 0, :], 64)
        h = (jnp.dot(oh_at, t1_ref[...], preferred_element_type=jnp.float32)
             + jnp.dot(oh_g, t2_ref[...], preferred_element_type=jnp.float32)
             + b_ref[...])
        h_ref[...] = h
        hs_ref[...] = (jnp.dot(h, wi_ref[...], preferred_element_type=jnp.float32)
                       + jnp.dot(oh_g, lt_ref[...], preferred_element_type=jnp.float32))
        hd_ref[...] = jnp.dot(h, wj_ref[...], preferred_element_type=jnp.float32)

    fixed = pl.BlockSpec(None, None)
    return pl.pallas_call(
        body,
        grid=(nb,),
        in_specs=[pl.BlockSpec((1, 1, _BN), lambda i: (i, 0, 0)),
                  pl.BlockSpec((1, 1, _BN), lambda i: (i, 0, 0)),
                  fixed, fixed, fixed, fixed, fixed, fixed],
        out_specs=[pl.BlockSpec((_BN, 128), lambda i: (i, 0)),
                   pl.BlockSpec((_BN, 128), lambda i: (i, 0)),
                   pl.BlockSpec((_BN, 128), lambda i: (i, 0))],
        out_shape=(jax.ShapeDtypeStruct((n, 128), jnp.float32),
                   jax.ShapeDtypeStruct((n, 128), jnp.float32),
                   jax.ShapeDtypeStruct((n, 128), jnp.float32)),
    )(at3, n2g3, t1, t2, b, latt0, wi, wj)


def _tc_dis(fdT, fmapT):
    e = fdT.shape[1]
    be = 2560
    nb = e // be

    def body(d_ref, f_ref, o_ref):
        ang_t = jnp.dot(f_ref[...], d_ref[...],
                        preferred_element_type=jnp.float32)
        row = lax.broadcasted_iota(jnp.int32, ang_t.shape, 0)
        dis_t = jnp.where(row < 32, jnp.sin(ang_t), jnp.cos(ang_t))
        o_ref[...] = dis_t.T

    fixed = pl.BlockSpec(None, None)
    return pl.pallas_call(
        body,
        grid=(nb,),
        in_specs=[pl.BlockSpec((4, be), lambda i: (0, i)),
                  fixed],
        out_specs=pl.BlockSpec((be, 64), lambda i: (i, 0)),
        out_shape=jax.ShapeDtypeStruct((e, 64), jnp.float32),
    )(fdT, fmapT)


def _tc_edge(ghs, ghd, dis, wdis, w2, b1, b2):
    e = ghs.shape[0]
    nb = e // _BE

    def body(s_ref, d_ref, x_ref, wd_ref, w2_ref, b1_ref, b2_ref, o_ref):
        pre = (s_ref[...] + d_ref[...] + b1_ref[...]
               + jnp.dot(x_ref[...], wd_ref[...],
                         preferred_element_type=jnp.float32))
        s1 = _silu(pre)
        z = jnp.dot(s1, w2_ref[...], preferred_element_type=jnp.float32) + b2_ref[...]
        o_ref[...] = _silu(z)

    fixed = pl.BlockSpec(None, None)
    return pl.pallas_call(
        body,
        grid=(nb,),
        in_specs=[pl.BlockSpec((_BE, 128), lambda i: (i, 0)),
                  pl.BlockSpec((_BE, 128), lambda i: (i, 0)),
                  pl.BlockSpec((_BE, 64), lambda i: (i, 0)),
                  fixed, fixed, fixed, fixed],
        out_specs=pl.BlockSpec((_BE, 128), lambda i: (i, 0)),
        out_shape=jax.ShapeDtypeStruct((e, 128), jnp.float32),
    )(ghs, ghd, dis, wdis, w2, b1, b2)


def _tc_rdenom(cnt2):
    npd = cnt2.shape[1]
    bn = 632

    def body(c_ref, o_ref):
        c = c_ref[0, :, 0:1] + c_ref[1, :, 0:1]
        o_ref[...] = jnp.broadcast_to(1.0 / jnp.maximum(c, 1.0), (bn, 8))

    return pl.pallas_call(
        body,
        grid=(npd // bn,),
        in_specs=[pl.BlockSpec((2, bn, 128), lambda i: (0, i, 0))],
        out_specs=pl.BlockSpec((bn, 8), lambda i: (i, 0)),
        out_shape=jax.ShapeDtypeStruct((npd, 8), jnp.float32),
    )(cnt2)


def _tc_node(h, agg2, rd, w1h, w1a, b1, w2, b2, nxt):
    n = h.shape[0]
    nb = n // _BN

    def body(h_ref, a_ref, c_ref, w1h_ref, w1a_ref, b1_ref, w2_ref, b2_ref,
             *rest):
        h = h_ref[...]
        agg = (a_ref[0] + a_ref[1]) * c_ref[:, 0:1]
        z1 = _silu(jnp.dot(h, w1h_ref[...], preferred_element_type=jnp.float32)
                   + jnp.dot(agg, w1a_ref[...], preferred_element_type=jnp.float32)
                   + b1_ref[...])
        z2 = _silu(jnp.dot(z1, w2_ref[...], preferred_element_type=jnp.float32)
                   + b2_ref[...])
        hn = h + z2
        if nxt is None:
            (ho_ref,) = rest
            ho_ref[...] = hn
        else:
            ng_ref, lt_ref, wi_ref, wj_ref, ho_ref, hs_ref, hd_ref = rest
            ho_ref[...] = hn
            oh_g = _onehot_from(ng_ref[0, 0, :], 64)
            hs_ref[...] = (jnp.dot(hn, wi_ref[...], preferred_element_type=jnp.float32)
                           + jnp.dot(oh_g, lt_ref[...], preferred_element_type=jnp.float32))
            hd_ref[...] = jnp.dot(hn, wj_ref[...], preferred_element_type=jnp.float32)

    fixed = pl.BlockSpec(None, None)
    in_specs = [pl.BlockSpec((_BN, 128), lambda i: (i, 0)),
                pl.BlockSpec((2, _BN, 128), lambda i: (0, i, 0)),
                pl.BlockSpec((_BN, 8), lambda i: (i, 0)),
                fixed, fixed, fixed, fixed, fixed]
    args = [h, agg2, rd, w1h, w1a, b1, w2, b2]
    out_specs = [pl.BlockSpec((_BN, 128), lambda i: (i, 0))]
    out_shape = [jax.ShapeDtypeStruct((n, 128), jnp.float32)]
    if nxt is not None:
        n2g3, latt_l, wi, wj = nxt
        in_specs += [pl.BlockSpec((1, 1, _BN), lambda i: (i, 0, 0)),
                     fixed, fixed, fixed]
        args += [n2g3, latt_l, wi, wj]
        out_specs += [pl.BlockSpec((_BN, 128), lambda i: (i, 0)),
                      pl.BlockSpec((_BN, 128), lambda i: (i, 0))]
        out_shape += [jax.ShapeDtypeStruct((n, 128), jnp.float32),
                      jax.ShapeDtypeStruct((n, 128), jnp.float32)]
    res = pl.pallas_call(
        body,
        grid=(nb,),
        in_specs=in_specs,
        out_specs=out_specs,
        out_shape=out_shape,
    )(*args)
    return res if nxt is not None else (res[0],)


def _tc_pool(h, n2g3, coordw, n):
    nb = n // _BN

    def body(h_ref, ng_ref, cw_ref, co_ref, gs_ref, gc_ref):
        i = pl.program_id(0)
        h = h_ref[...]
        co_ref[...] = jnp.dot(h, cw_ref[...], preferred_element_type=jnp.float32)
        ids = ng_ref[0, 0, :]
        oht = (lax.broadcasted_iota(jnp.int32, (64, _BN), 0) == ids[None, :]
               ).astype(jnp.float32)

        @pl.when(i == 0)
        def _():
            gs_ref[...] = jnp.zeros_like(gs_ref)
            gc_ref[...] = jnp.zeros_like(gc_ref)

        gs_ref[...] += jnp.dot(oht, h, preferred_element_type=jnp.float32)
        gc_ref[...] += jnp.dot(oht, jnp.ones((_BN, 128), jnp.float32),
                               preferred_element_type=jnp.float32)

    fixed = pl.BlockSpec(None, None)
    return pl.pallas_call(
        body,
        grid=(nb,),
        in_specs=[pl.BlockSpec((_BN, 128), lambda i: (i, 0)),
                  pl.BlockSpec((1, 1, _BN), lambda i: (i, 0, 0)),
                  fixed],
        out_specs=[pl.BlockSpec((_BN, 8), lambda i: (i, 0)),
                   pl.BlockSpec((64, 128), lambda i: (0, 0)),
                   pl.BlockSpec((64, 128), lambda i: (0, 0))],
        out_shape=(jax.ShapeDtypeStruct((n, 8), jnp.float32),
                   jax.ShapeDtypeStruct((64, 128), jnp.float32),
                   jax.ShapeDtypeStruct((64, 128), jnp.float32)),
    )(h, n2g3, coordw)


def _tc_lattice(gsum, gcnt, latw, lat9):
    g = lat9.shape[0]

    def body(gs_ref, gc_ref, w_ref, l_ref, o_ref):
        gf = gs_ref[...] / jnp.maximum(gc_ref[...], 1.0)
        gl = jnp.dot(gf, w_ref[...], preferred_element_type=jnp.float32)
        l = l_ref[...]
        cols = []
        for i in range(3):
            for kk in range(3):
                acc = gl[:, 3 * i + 0] * l[:, 0 + kk]
                acc = acc + gl[:, 3 * i + 1] * l[:, 3 + kk]
                acc = acc + gl[:, 3 * i + 2] * l[:, 6 + kk]
                cols.append(acc)
        for _ in range(7):
            cols.append(jnp.zeros((g,), jnp.float32))
        o_ref[...] = jnp.stack(cols, axis=1)

    return pl.pallas_call(
        body,
        out_shape=jax.ShapeDtypeStruct((g, 16), jnp.float32),
    )(gsum, gcnt, latw, lat9)


def kernel(atom_types, frac_coords, lattices, edge_index, node2graph, t, params):
    n = atom_types.shape[0]
    e = edge_index.shape[1]
    g = lattices.shape[0]
    hid = 128

    src = edge_index[0].astype(jnp.int32).reshape(1, e)
    dst = edge_index[1].astype(jnp.int32).reshape(1, e)
    at3 = atom_types.astype(jnp.int32).reshape(n // _BN, 1, _BN)
    n2g3 = node2graph.astype(jnp.int32).reshape(n // _BN, 1, _BN)

    p = params
    layers = p["layers"]

    wtop = p["atom_latent_W"][:hid]
    wbot = p["atom_latent_W"][hid:]
    b_al = p["atom_latent_b"].reshape(1, hid)
    emb_pad = jnp.zeros((128, hid), jnp.float32).at[:p["node_emb"].shape[0]].set(
        p["node_emb"])
    lat9 = jnp.pad(lattices.reshape(g, 9), ((0, 0), (0, 7)))
    wlat_all = jnp.concatenate(
        [jnp.pad(lp["eW1"][2 * hid:2 * hid + 9], ((0, 7), (0, 0)))
         for lp in layers], axis=0)
    wi = [lp["eW1"][:hid] for lp in layers]
    wj = [lp["eW1"][hid:2 * hid] for lp in layers]
    wdis = []
    for lp in layers:
        wd = lp["eW1"][2 * hid + 9:]
        wdis.append(jnp.concatenate([
            wd[:30], jnp.zeros((2, hid), jnp.float32),
            wd[30:], jnp.zeros((2, hid), jnp.float32)], axis=0))
    eb1 = [lp["eb1"].reshape(1, hid) for lp in layers]
    ew2 = [lp["eW2"] for lp in layers]
    eb2 = [lp["eb2"].reshape(1, hid) for lp in layers]
    nw1h = [lp["nW1"][:hid] for lp in layers]
    nw1a = [lp["nW1"][hid:] for lp in layers]
    nb1 = [lp["nb1"].reshape(1, hid) for lp in layers]
    nw2 = [lp["nW2"] for lp in layers]
    nb2 = [lp["nb2"].reshape(1, hid) for lp in layers]
    coordw = jnp.pad(p["coord_W"], ((0, 0), (0, 5)))
    latw = jnp.pad(p["lattice_W"], ((0, 0), (0, 7)))

    freqs = 2.0 * math.pi * np.arange(NFREQ, dtype=np.float32)
    fmap_np = np.zeros((64, 4), np.float32)
    for j in range(3):
        for f in range(NFREQ):
            fmap_np[j * NFREQ + f, j] = freqs[f]
            fmap_np[32 + j * NFREQ + f, j] = freqs[f]
    fmap_t = jnp.asarray(fmap_np)

    frac_flat = jnp.pad(frac_coords, ((0, 0), (0, 1))).reshape(-1)
    npad = ((n + 8 * _SC_TILES - 1) // (8 * _SC_TILES)) * (8 * _SC_TILES)
    zeros_nd = jnp.zeros((npad, hid), jnp.float32)
    zeros_nc = jnp.zeros((npad, hid), jnp.float32)
    ones_w = jnp.ones((_SC_W, hid), jnp.float32)

    latt_all, t1, t2 = _tc_tables(lat9, wlat_all, emb_pad, wtop, t, wbot)
    cnt2 = _sc_counts(src, ones_w, zeros_nc)
    rd = _tc_rdenom(cnt2)
    fdT = _sc_fdiff(frac_flat, src, dst)
    dis = _tc_dis(fdT, fmap_t)
    h, hs, hd = _tc_h0(at3, n2g3, t1, t2, b_al,
                       lax.slice_in_dim(latt_all, 0, g), wi[0], wj[0], n)

    for l in range(4):
        ghs, ghd = _sc_gather_pairs(hs, src, hd, dst)
        ef = _tc_edge(ghs, ghd, dis, wdis[l], ew2[l], eb1[l], eb2[l])
        agg2 = _sc_scatter_rows(ef, src, zeros_nd)
        if l < 3:
            nxt = (n2g3, lax.slice_in_dim(latt_all, (l + 1) * g, (l + 2) * g),
                   wi[l + 1], wj[l + 1])
            h, hs, hd = _tc_node(h, agg2, rd, nw1h[l], nw1a[l], nb1[l],
                                 nw2[l], nb2[l], nxt)
        else:
            (h,) = _tc_node(h, agg2, rd, nw1h[l], nw1a[l], nb1[l],
                            nw2[l], nb2[l], None)

    coord8, gsum, gcnt = _tc_pool(h, n2g3, coordw, n)
    lo16 = _tc_lattice(gsum, gcnt, latw, lat9)

    coord_out = coord8[:, :3]
    lattice_out = lo16[:, :9].reshape(g, 3, 3)
    return lattice_out, coord_out


# speedup vs baseline: 11.1551x; 1.0861x over previous
import dataclasses
import functools
import math

import jax
import jax.numpy as jnp
import numpy as np
from jax import lax
from jax.experimental import pallas as pl
from jax.experimental.pallas import tpu as pltpu
from jax.experimental.pallas import tpu_sc as plsc

NFREQ = 10


_SC_CORES = 2
_SC_TILES = 16
_SC_W = 128


def _sc_mesh():
    return plsc.VectorSubcoreMesh(
        core_axis_name="c", subcore_axis_name="s",
        num_cores=_SC_CORES, num_subcores=_SC_TILES)


def _sc_gather_add(t1, i1, t2, i2):
    n, d = t1.shape
    e = i1.shape[1]
    w = _SC_W

    @functools.partial(
        pl.kernel,
        out_type=jax.ShapeDtypeStruct((e, d), jnp.float32),
        mesh=_sc_mesh())
    def k(t1_hbm, i1_hbm, t2_hbm, i2_hbm, o_hbm):
        def body(i1_v, i2_v, o_v):
            pltpu.sync_copy(t1_hbm.at[i1_v.at[0]], o_v)
            pltpu.sync_copy(t2_hbm.at[i2_v.at[0]], o_v, add=True)

        pltpu.emit_pipeline(
            body,
            grid=(e // w,),
            in_specs=[pl.BlockSpec((1, w), lambda i: (0, i)),
                      pl.BlockSpec((1, w), lambda i: (0, i))],
            out_specs=[pl.BlockSpec((w, d), lambda i: (i, 0))],
            core_axis_name=("c", "s"),
            dimension_semantics=(pltpu.PARALLEL,),
        )(i1_hbm, i2_hbm, o_hbm)

    return k(t1, i1, t2, i2)


def _sc_fdiff(frac_flat, src, dst):
    e = src.shape[1]
    w = _SC_W
    nflat = frac_flat.shape[0]

    cp = pltpu.CompilerParams()
    if "needs_layout_passes" in pltpu.CompilerParams.__dataclass_fields__:
        cp = dataclasses.replace(cp, needs_layout_passes=False)

    @functools.partial(
        pl.kernel,
        out_type=jax.ShapeDtypeStruct((4, e), jnp.float32),
        mesh=_sc_mesh(),
        compiler_params=cp,
        scratch_types=[pltpu.VMEM((nflat,), jnp.float32)])
    def k(f_hbm, s_hbm, d_hbm, o_hbm, tbl):
        pltpu.sync_copy(f_hbm, tbl)

        def body(s_v, d_v, o_v):
            for gi in range(w // 16):
                sl = pl.ds(gi * 16, 16)
                s16 = s_v[0, sl] * 4
                d16 = d_v[0, sl] * 4
                for c in range(3):
                    fs = plsc.load_gather(tbl, [s16 + c])
                    fd = plsc.load_gather(tbl, [d16 + c])
                    o_v[c, sl] = fd - fs
                o_v[3, sl] = jnp.zeros((16,), jnp.float32)

        pltpu.emit_pipeline(
            body,
            grid=(e // w,),
            in_specs=[pl.BlockSpec((1, w), lambda i: (0, i)),
                      pl.BlockSpec((1, w), lambda i: (0, i))],
            out_specs=[pl.BlockSpec((4, w), lambda i: (0, i))],
            core_axis_name=("c", "s"),
            dimension_semantics=(pltpu.PARALLEL,),
        )(s_hbm, d_hbm, o_hbm)

    return k(frac_flat, src, dst)


def _sc_scatter_rows(vals, idx, zeros):
    e, d = vals.shape
    n = zeros.shape[0]
    w = _SC_W
    rows = n // _SC_TILES

    @functools.partial(
        pl.kernel,
        out_type=jax.ShapeDtypeStruct((_SC_CORES, n, d), jnp.float32),
        mesh=_sc_mesh(),
        scratch_types=[pltpu.VMEM_SHARED((n, d), jnp.float32)])
    def k(v_hbm, i_hbm, z_hbm, o_hbm, acc):
        cid = lax.axis_index("c")
        sid = lax.axis_index("s")
        pltpu.sync_copy(z_hbm.at[pl.ds(sid * rows, rows)],
                        acc.at[pl.ds(sid * rows, rows)])
        plsc.subcore_barrier()

        def body(v_v, i_v):
            pltpu.sync_copy(v_v, acc.at[i_v.at[0]], add=True)

        pltpu.emit_pipeline(
            body,
            grid=(e // w,),
            in_specs=[pl.BlockSpec((w, d), lambda i: (i, 0)),
                      pl.BlockSpec((1, w), lambda i: (0, i))],
            out_specs=[],
            core_axis_name=("c", "s"),
            dimension_semantics=(pltpu.PARALLEL,),
        )(v_hbm, i_hbm)

        plsc.subcore_barrier()
        pltpu.sync_copy(acc.at[pl.ds(sid * rows, rows)],
                        o_hbm.at[cid].at[pl.ds(sid * rows, rows)])

    return k(vals, idx, zeros)


def _sc_counts(idx, ones, zeros):
    e = idx.shape[1]
    n, dc = zeros.shape
    w = _SC_W
    rows = n // _SC_TILES

    @functools.partial(
        pl.kernel,
        out_type=jax.ShapeDtypeStruct((_SC_CORES, n, dc), jnp.float32),
        mesh=_sc_mesh(),
        scratch_types=[pltpu.VMEM((w, dc), jnp.float32),
                       pltpu.VMEM_SHARED((n, dc), jnp.float32)])
    def k(i_hbm, one_hbm, z_hbm, o_hbm, ones_v, acc):
        cid = lax.axis_index("c")
        sid = lax.axis_index("s")
        pltpu.sync_copy(one_hbm, ones_v)
        pltpu.sync_copy(z_hbm.at[pl.ds(sid * rows, rows)],
                        acc.at[pl.ds(sid * rows, rows)])
        plsc.subcore_barrier()

        def body(i_v):
            pltpu.sync_copy(ones_v, acc.at[i_v.at[0]], add=True)

        pltpu.emit_pipeline(
            body,
            grid=(e // w,),
            in_specs=[pl.BlockSpec((1, w), lambda i: (0, i))],
            out_specs=[],
            core_axis_name=("c", "s"),
            dimension_semantics=(pltpu.PARALLEL,),
        )(i_hbm)

        plsc.subcore_barrier()
        pltpu.sync_copy(acc.at[pl.ds(sid * rows, rows)],
                        o_hbm.at[cid].at[pl.ds(sid * rows, rows)])

    return k(idx, ones, zeros)


_BN = 1000
_BE = 4000


def _silu(x):
    return x * jax.nn.sigmoid(x)


def _onehot_from(ids, nclass):
    return (ids[:, None] == lax.broadcasted_iota(jnp.int32, (ids.shape[0], nclass), 1)
            ).astype(jnp.float32)


def _tc_tables(lat9, wlat_all, emb_pad, wtop, t, wbot):
    g = lat9.shape[0]

    def body(l_ref, wl_ref, e_ref, wt_ref, t_ref, wb_ref,
             latt_ref, t1_ref, t2_ref):
        l = l_ref[...]
        cols = []
        for i in range(3):
            for kk in range(3):
                acc = l[:, 3 * i + 0] * l[:, 3 * kk + 0]
                acc = acc + l[:, 3 * i + 1] * l[:, 3 * kk + 1]
                acc = acc + l[:, 3 * i + 2] * l[:, 3 * kk + 2]
                cols.append(acc)
        for _ in range(7):
            cols.append(jnp.zeros((g,), jnp.float32))
        lat_ip = jnp.stack(cols, axis=1)
        for layer in range(4):
            wl = wl_ref[pl.ds(16 * layer, 16), :]
            latt_ref[pl.ds(g * layer, g), :] = jnp.dot(
                lat_ip, wl, preferred_element_type=jnp.float32)
        t1_ref[...] = jnp.dot(e_ref[...], wt_ref[...],
                              preferred_element_type=jnp.float32)
        t2_ref[...] = jnp.dot(t_ref[...], wb_ref[...],
                              preferred_element_type=jnp.float32)

    return pl.pallas_call(
        body,
        out_shape=(jax.ShapeDtypeStruct((4 * g, 128), jnp.float32),
                   jax.ShapeDtypeStruct((128, 128), jnp.float32),
                   jax.ShapeDtypeStruct((g, 128), jnp.float32)),
    )(lat9, wlat_all, emb_pad, wtop, t, wbot)


def _tc_h0(at3, n2g3, t1, t2, b, latt0, wi, wj, n):
    nb = n // _BN

    def body(at_ref, ng_ref, t1_ref, t2_ref, b_ref, lt_ref, wi_ref, wj_ref,
             h_ref, hs_ref, hd_ref):
        oh_at = _onehot_from(at_ref[0, 0, :], 128)
        oh_g = _onehot_from(ng_ref[0, 0, :], 64)
        h = (jnp.dot(oh_at, t1_ref[...], preferred_element_type=jnp.float32)
             + jnp.dot(oh_g, t2_ref[...], preferred_element_type=jnp.float32)
             + b_ref[...])
        h_ref[...] = h
        hs_ref[...] = (jnp.dot(h, wi_ref[...], preferred_element_type=jnp.float32)
                       + jnp.dot(oh_g, lt_ref[...], preferred_element_type=jnp.float32))
        hd_ref[...] = jnp.dot(h, wj_ref[...], preferred_element_type=jnp.float32)

    fixed = pl.BlockSpec(None, None)
    return pl.pallas_call(
        body,
        grid=(nb,),
        in_specs=[pl.BlockSpec((1, 1, _BN), lambda i: (i, 0, 0)),
                  pl.BlockSpec((1, 1, _BN), lambda i: (i, 0, 0)),
                  fixed, fixed, fixed, fixed, fixed, fixed],
        out_specs=[pl.BlockSpec((_BN, 128), lambda i: (i, 0)),
                   pl.BlockSpec((_BN, 128), lambda i: (i, 0)),
                   pl.BlockSpec((_BN, 128), lambda i: (i, 0))],
        out_shape=(jax.ShapeDtypeStruct((n, 128), jnp.float32),
                   jax.ShapeDtypeStruct((n, 128), jnp.float32),
                   jax.ShapeDtypeStruct((n, 128), jnp.float32)),
    )(at3, n2g3, t1, t2, b, latt0, wi, wj)


def _tc_dis(fdT, fmapT):
    e = fdT.shape[1]
    be = 2560
    nb = e // be

    def body(d_ref, f_ref, o_ref):
        ang_t = jnp.dot(f_ref[...], d_ref[...],
                        preferred_element_type=jnp.float32)
        row = lax.broadcasted_iota(jnp.int32, ang_t.shape, 0)
        dis_t = jnp.where(row < 32, jnp.sin(ang_t), jnp.cos(ang_t))
        o_ref[...] = dis_t.T

    fixed = pl.BlockSpec(None, None)
    return pl.pallas_call(
        body,
        grid=(nb,),
        in_specs=[pl.BlockSpec((4, be), lambda i: (0, i)),
                  fixed],
        out_specs=pl.BlockSpec((be, 64), lambda i: (i, 0)),
        out_shape=jax.ShapeDtypeStruct((e, 64), jnp.float32),
    )(fdT, fmapT)


def _tc_edge(gsd, dis, wdis, w2, b1, b2):
    e = gsd.shape[0]
    nb = e // _BE

    def body(s_ref, x_ref, wd_ref, w2_ref, b1_ref, b2_ref, o_ref):
        pre = (s_ref[...].astype(jnp.float32) + b1_ref[...]
               + jnp.dot(x_ref[...], wd_ref[...],
                         preferred_element_type=jnp.float32))
        s1 = _silu(pre)
        z = jnp.dot(s1, w2_ref[...], preferred_element_type=jnp.float32) + b2_ref[...]
        o_ref[...] = _silu(z)

    fixed = pl.BlockSpec(None, None)
    return pl.pallas_call(
        body,
        grid=(nb,),
        in_specs=[pl.BlockSpec((_BE, 128), lambda i: (i, 0)),
                  pl.BlockSpec((_BE, 64), lambda i: (i, 0)),
                  fixed, fixed, fixed, fixed],
        out_specs=pl.BlockSpec((_BE, 128), lambda i: (i, 0)),
        out_shape=jax.ShapeDtypeStruct((e, 128), jnp.float32),
    )(gsd, dis, wdis, w2, b1, b2)


def _tc_rdenom(cnt2):
    npd = cnt2.shape[1]
    bn = 632

    def body(c_ref, o_ref):
        c = c_ref[0, :, 0:1] + c_ref[1, :, 0:1]
        o_ref[...] = jnp.broadcast_to(1.0 / jnp.maximum(c, 1.0), (bn, 8))

    return pl.pallas_call(
        body,
        grid=(npd // bn,),
        in_specs=[pl.BlockSpec((2, bn, 128), lambda i: (0, i, 0))],
        out_specs=pl.BlockSpec((bn, 8), lambda i: (i, 0)),
        out_shape=jax.ShapeDtypeStruct((npd, 8), jnp.float32),
    )(cnt2)


def _tc_node(h, agg2, rd, w1h, w1a, b1, w2, b2, nxt):
    n = h.shape[0]
    nb = n // _BN

    def body(h_ref, a_ref, c_ref, w1h_ref, w1a_ref, b1_ref, w2_ref, b2_ref,
             *rest):
        h = h_ref[...]
        agg = (a_ref[0] + a_ref[1]) * c_ref[:, 0:1]
        z1 = _silu(jnp.dot(h, w1h_ref[...], preferred_element_type=jnp.float32)
                   + jnp.dot(agg, w1a_ref[...], preferred_element_type=jnp.float32)
                   + b1_ref[...])
        z2 = _silu(jnp.dot(z1, w2_ref[...], preferred_element_type=jnp.float32)
                   + b2_ref[...])
        hn = h + z2
        if nxt is None:
            (ho_ref,) = rest
            ho_ref[...] = hn
        else:
            ng_ref, lt_ref, wi_ref, wj_ref, ho_ref, hs_ref, hd_ref = rest
            ho_ref[...] = hn
            oh_g = _onehot_from(ng_ref[0, 0, :], 64)
            hs_ref[...] = (jnp.dot(hn, wi_ref[...], preferred_element_type=jnp.float32)
                           + jnp.dot(oh_g, lt_ref[...], preferred_element_type=jnp.float32))
            hd_ref[...] = jnp.dot(hn, wj_ref[...], preferred_element_type=jnp.float32)

    fixed = pl.BlockSpec(None, None)
    in_specs = [pl.BlockSpec((_BN, 128), lambda i: (i, 0)),
                pl.BlockSpec((2, _BN, 128), lambda i: (0, i, 0)),
                pl.BlockSpec((_BN, 8), lambda i: (i, 0)),
                fixed, fixed, fixed, fixed, fixed]
    args = [h, agg2, rd, w1h, w1a, b1, w2, b2]
    out_specs = [pl.BlockSpec((_BN, 128), lambda i: (i, 0))]
    out_shape = [jax.ShapeDtypeStruct((n, 128), jnp.float32)]
    if nxt is not None:
        n2g3, latt_l, wi, wj = nxt
        in_specs += [pl.BlockSpec((1, 1, _BN), lambda i: (i, 0, 0)),
                     fixed, fixed, fixed]
        args += [n2g3, latt_l, wi, wj]
        out_specs += [pl.BlockSpec((_BN, 128), lambda i: (i, 0)),
                      pl.BlockSpec((_BN, 128), lambda i: (i, 0))]
        out_shape += [jax.ShapeDtypeStruct((n, 128), jnp.float32),
                      jax.ShapeDtypeStruct((n, 128), jnp.float32)]
    res = pl.pallas_call(
        body,
        grid=(nb,),
        in_specs=in_specs,
        out_specs=out_specs,
        out_shape=out_shape,
    )(*args)
    return res if nxt is not None else (res[0],)


def _tc_pool(h, n2g3, coordw, n):
    nb = n // _BN

    def body(h_ref, ng_ref, cw_ref, co_ref, gs_ref, gc_ref):
        i = pl.program_id(0)
        h = h_ref[...]
        co_ref[...] = jnp.dot(h, cw_ref[...], preferred_element_type=jnp.float32)
        ids = ng_ref[0, 0, :]
        oht = (lax.broadcasted_iota(jnp.int32, (64, _BN), 0) == ids[None, :]
               ).astype(jnp.float32)

        @pl.when(i == 0)
        def _():
            gs_ref[...] = jnp.zeros_like(gs_ref)
            gc_ref[...] = jnp.zeros_like(gc_ref)

        gs_ref[...] += jnp.dot(oht, h, preferred_element_type=jnp.float32)
        gc_ref[...] += jnp.dot(oht, jnp.ones((_BN, 128), jnp.float32),
                               preferred_element_type=jnp.float32)

    fixed = pl.BlockSpec(None, None)
    return pl.pallas_call(
        body,
        grid=(nb,),
        in_specs=[pl.BlockSpec((_BN, 128), lambda i: (i, 0)),
                  pl.BlockSpec((1, 1, _BN), lambda i: (i, 0, 0)),
                  fixed],
        out_specs=[pl.BlockSpec((_BN, 8), lambda i: (i, 0)),
                   pl.BlockSpec((64, 128), lambda i: (0, 0)),
                   pl.BlockSpec((64, 128), lambda i: (0, 0))],
        out_shape=(jax.ShapeDtypeStruct((n, 8), jnp.float32),
                   jax.ShapeDtypeStruct((64, 128), jnp.float32),
                   jax.ShapeDtypeStruct((64, 128), jnp.float32)),
    )(h, n2g3, coordw)


def _tc_lattice(gsum, gcnt, latw, lat9):
    g = lat9.shape[0]

    def body(gs_ref, gc_ref, w_ref, l_ref, o_ref):
        gf = gs_ref[...] / jnp.maximum(gc_ref[...], 1.0)
        gl = jnp.dot(gf, w_ref[...], preferred_element_type=jnp.float32)
        l = l_ref[...]
        cols = []
        for i in range(3):
            for kk in range(3):
                acc = gl[:, 3 * i + 0] * l[:, 0 + kk]
                acc = acc + gl[:, 3 * i + 1] * l[:, 3 + kk]
                acc = acc + gl[:, 3 * i + 2] * l[:, 6 + kk]
                cols.append(acc)
        for _ in range(7):
            cols.append(jnp.zeros((g,), jnp.float32))
        o_ref[...] = jnp.stack(cols, axis=1)

    return pl.pallas_call(
        body,
        out_shape=jax.ShapeDtypeStruct((g, 16), jnp.float32),
    )(gsum, gcnt, latw, lat9)


def kernel(atom_types, frac_coords, lattices, edge_index, node2graph, t, params):
    n = atom_types.shape[0]
    e = edge_index.shape[1]
    g = lattices.shape[0]
    hid = 128

    src = edge_index[0].astype(jnp.int32).reshape(1, e)
    dst = edge_index[1].astype(jnp.int32).reshape(1, e)
    at3 = atom_types.astype(jnp.int32).reshape(n // _BN, 1, _BN)
    n2g3 = node2graph.astype(jnp.int32).reshape(n // _BN, 1, _BN)

    p = params
    layers = p["layers"]

    wtop = p["atom_latent_W"][:hid]
    wbot = p["atom_latent_W"][hid:]
    b_al = p["atom_latent_b"].reshape(1, hid)
    emb_pad = jnp.zeros((128, hid), jnp.float32).at[:p["node_emb"].shape[0]].set(
        p["node_emb"])
    lat9 = jnp.pad(lattices.reshape(g, 9), ((0, 0), (0, 7)))
    wlat_all = jnp.concatenate(
        [jnp.pad(lp["eW1"][2 * hid:2 * hid + 9], ((0, 7), (0, 0)))
         for lp in layers], axis=0)
    wi = [lp["eW1"][:hid] for lp in layers]
    wj = [lp["eW1"][hid:2 * hid] for lp in layers]
    wdis = []
    for lp in layers:
        wd = lp["eW1"][2 * hid + 9:]
        wdis.append(jnp.concatenate([
            wd[:30], jnp.zeros((2, hid), jnp.float32),
            wd[30:], jnp.zeros((2, hid), jnp.float32)], axis=0))
    eb1 = [lp["eb1"].reshape(1, hid) for lp in layers]
    ew2 = [lp["eW2"] for lp in layers]
    eb2 = [lp["eb2"].reshape(1, hid) for lp in layers]
    nw1h = [lp["nW1"][:hid] for lp in layers]
    nw1a = [lp["nW1"][hid:] for lp in layers]
    nb1 = [lp["nb1"].reshape(1, hid) for lp in layers]
    nw2 = [lp["nW2"] for lp in layers]
    nb2 = [lp["nb2"].reshape(1, hid) for lp in layers]
    coordw = jnp.pad(p["coord_W"], ((0, 0), (0, 5)))
    latw = jnp.pad(p["lattice_W"], ((0, 0), (0, 7)))

    freqs = 2.0 * math.pi * np.arange(NFREQ, dtype=np.float32)
    fmap_np = np.zeros((64, 4), np.float32)
    for j in range(3):
        for f in range(NFREQ):
            fmap_np[j * NFREQ + f, j] = freqs[f]
            fmap_np[32 + j * NFREQ + f, j] = freqs[f]
    fmap_t = jnp.asarray(fmap_np)

    frac_flat = jnp.pad(frac_coords, ((0, 0), (0, 1))).reshape(-1)
    npad = ((n + 8 * _SC_TILES - 1) // (8 * _SC_TILES)) * (8 * _SC_TILES)
    zeros_nd = jnp.zeros((npad, hid), jnp.float32)
    zeros_nc = jnp.zeros((npad, hid), jnp.float32)
    ones_w = jnp.ones((_SC_W, hid), jnp.float32)

    latt_all, t1, t2 = _tc_tables(lat9, wlat_all, emb_pad, wtop, t, wbot)
    cnt2 = _sc_counts(src, ones_w, zeros_nc)
    rd = _tc_rdenom(cnt2)
    fdT = _sc_fdiff(frac_flat, src, dst)
    dis = _tc_dis(fdT, fmap_t)
    h, hs, hd = _tc_h0(at3, n2g3, t1, t2, b_al,
                       lax.slice_in_dim(latt_all, 0, g), wi[0], wj[0], n)

    for l in range(4):
        gsd = _sc_gather_add(hs, src, hd, dst)
        ef = _tc_edge(gsd, dis, wdis[l], ew2[l], eb1[l], eb2[l])
        agg2 = _sc_scatter_rows(ef, src, zeros_nd)
        if l < 3:
            nxt = (n2g3, lax.slice_in_dim(latt_all, (l + 1) * g, (l + 2) * g),
                   wi[l + 1], wj[l + 1])
            h, hs, hd = _tc_node(h, agg2, rd, nw1h[l], nw1a[l], nb1[l],
                                 nw2[l], nb2[l], nxt)
        else:
            (h,) = _tc_node(h, agg2, rd, nw1h[l], nw1a[l], nb1[l],
                            nw2[l], nb2[l], None)

    coord8, gsum, gcnt = _tc_pool(h, n2g3, coordw, n)
    lo16 = _tc_lattice(gsum, gcnt, latw, lat9)

    coord_out = coord8[:, :3]
    lattice_out = lo16[:, :9].reshape(g, 3, 3)
    return lattice_out, coord_out


# speedup vs baseline: 11.9142x; 1.0680x over previous
import dataclasses
import functools
import math

import jax
import jax.numpy as jnp
import numpy as np
from jax import lax
from jax.experimental import pallas as pl
from jax.experimental.pallas import tpu as pltpu
from jax.experimental.pallas import tpu_sc as plsc

NFREQ = 10


_SC_CORES = 2
_SC_TILES = 16
_SC_W = 128


def _sc_mesh():
    return plsc.VectorSubcoreMesh(
        core_axis_name="c", subcore_axis_name="s",
        num_cores=_SC_CORES, num_subcores=_SC_TILES)


def _sc_gather_add(t1, i1, t2, i2):
    n, d = t1.shape
    e = i1.shape[1]
    w = _SC_W

    @functools.partial(
        pl.kernel,
        out_type=jax.ShapeDtypeStruct((e, d), jnp.float32),
        mesh=_sc_mesh())
    def k(t1_hbm, i1_hbm, t2_hbm, i2_hbm, o_hbm):
        def body(i1_v, i2_v, o_v):
            pltpu.sync_copy(t1_hbm.at[i1_v.at[0]], o_v)
            pltpu.sync_copy(t2_hbm.at[i2_v.at[0]], o_v, add=True)

        pltpu.emit_pipeline(
            body,
            grid=(e // w,),
            in_specs=[pl.BlockSpec((1, w), lambda i: (0, i)),
                      pl.BlockSpec((1, w), lambda i: (0, i))],
            out_specs=[pl.BlockSpec((w, d), lambda i: (i, 0))],
            core_axis_name=("c", "s"),
            dimension_semantics=(pltpu.PARALLEL,),
        )(i1_hbm, i2_hbm, o_hbm)

    return k(t1, i1, t2, i2)


def _sc_fdiff(frac_flat, src, dst):
    e = src.shape[1]
    w = _SC_W
    nflat = frac_flat.shape[0]

    cp = pltpu.CompilerParams()
    if "needs_layout_passes" in pltpu.CompilerParams.__dataclass_fields__:
        cp = dataclasses.replace(cp, needs_layout_passes=False)

    @functools.partial(
        pl.kernel,
        out_type=jax.ShapeDtypeStruct((4, e), jnp.float32),
        mesh=_sc_mesh(),
        compiler_params=cp,
        scratch_types=[pltpu.VMEM((nflat,), jnp.float32)])
    def k(f_hbm, s_hbm, d_hbm, o_hbm, tbl):
        pltpu.sync_copy(f_hbm, tbl)

        def body(s_v, d_v, o_v):
            for gi in range(w // 16):
                sl = pl.ds(gi * 16, 16)
                s16 = s_v[0, sl] * 4
                d16 = d_v[0, sl] * 4
                for c in range(3):
                    fs = plsc.load_gather(tbl, [s16 + c])
                    fd = plsc.load_gather(tbl, [d16 + c])
                    o_v[c, sl] = fd - fs
                o_v[3, sl] = jnp.zeros((16,), jnp.float32)

        pltpu.emit_pipeline(
            body,
            grid=(e // w,),
            in_specs=[pl.BlockSpec((1, w), lambda i: (0, i)),
                      pl.BlockSpec((1, w), lambda i: (0, i))],
            out_specs=[pl.BlockSpec((4, w), lambda i: (0, i))],
            core_axis_name=("c", "s"),
            dimension_semantics=(pltpu.PARALLEL,),
        )(s_hbm, d_hbm, o_hbm)

    return k(frac_flat, src, dst)


def _sc_scatter_rows(vals, idx, init):
    e, d = vals.shape
    n = init.shape[1]
    w = _SC_W
    rows = n // _SC_TILES

    @functools.partial(
        pl.kernel,
        out_type=jax.ShapeDtypeStruct((_SC_CORES, n, d), jnp.float32),
        mesh=_sc_mesh(),
        scratch_types=[pltpu.VMEM_SHARED((n, d), jnp.float32)])
    def k(v_hbm, i_hbm, z_hbm, o_hbm, acc):
        cid = lax.axis_index("c")
        sid = lax.axis_index("s")
        pltpu.sync_copy(z_hbm.at[cid].at[pl.ds(sid * rows, rows)],
                        acc.at[pl.ds(sid * rows, rows)])
        plsc.subcore_barrier()

        def body(v_v, i_v):
            pltpu.sync_copy(v_v, acc.at[i_v.at[0]], add=True)

        pltpu.emit_pipeline(
            body,
            grid=(e // w,),
            in_specs=[pl.BlockSpec((w, d), lambda i: (i, 0)),
                      pl.BlockSpec((1, w), lambda i: (0, i))],
            out_specs=[],
            core_axis_name=("c", "s"),
            dimension_semantics=(pltpu.PARALLEL,),
        )(v_hbm, i_hbm)

        plsc.subcore_barrier()
        pltpu.sync_copy(acc.at[pl.ds(sid * rows, rows)],
                        o_hbm.at[cid].at[pl.ds(sid * rows, rows)])

    return k(vals, idx, init)


def _sc_counts(idx, ones, zeros):
    e = idx.shape[1]
    n, dc = zeros.shape
    w = _SC_W
    rows = n // _SC_TILES

    @functools.partial(
        pl.kernel,
        out_type=jax.ShapeDtypeStruct((_SC_CORES, n, dc), jnp.float32),
        mesh=_sc_mesh(),
        scratch_types=[pltpu.VMEM((w, dc), jnp.float32),
                       pltpu.VMEM_SHARED((n, dc), jnp.float32)])
    def k(i_hbm, one_hbm, z_hbm, o_hbm, ones_v, acc):
        cid = lax.axis_index("c")
        sid = lax.axis_index("s")
        pltpu.sync_copy(one_hbm, ones_v)
        pltpu.sync_copy(z_hbm.at[pl.ds(sid * rows, rows)],
                        acc.at[pl.ds(sid * rows, rows)])
        plsc.subcore_barrier()

        def body(i_v):
            pltpu.sync_copy(ones_v, acc.at[i_v.at[0]], add=True)

        pltpu.emit_pipeline(
            body,
            grid=(e // w,),
            in_specs=[pl.BlockSpec((1, w), lambda i: (0, i))],
            out_specs=[],
            core_axis_name=("c", "s"),
            dimension_semantics=(pltpu.PARALLEL,),
        )(i_hbm)

        plsc.subcore_barrier()
        pltpu.sync_copy(acc.at[pl.ds(sid * rows, rows)],
                        o_hbm.at[cid].at[pl.ds(sid * rows, rows)])

    return k(idx, ones, zeros)


_BN = 1000
_BE = 4000


def _silu(x):
    return x * jax.nn.sigmoid(x)


def _onehot_from(ids, nclass):
    return (ids[:, None] == lax.broadcasted_iota(jnp.int32, (ids.shape[0], nclass), 1)
            ).astype(jnp.float32)


def _tc_tables(lat9, wlat_all, emb_pad, wtop, t, wbot):
    g = lat9.shape[0]

    def body(l_ref, wl_ref, e_ref, wt_ref, t_ref, wb_ref,
             latt_ref, t1_ref, t2_ref):
        l = l_ref[...]
        cols = []
        for i in range(3):
            for kk in range(3):
                acc = l[:, 3 * i + 0] * l[:, 3 * kk + 0]
                acc = acc + l[:, 3 * i + 1] * l[:, 3 * kk + 1]
                acc = acc + l[:, 3 * i + 2] * l[:, 3 * kk + 2]
                cols.append(acc)
        for _ in range(7):
            cols.append(jnp.zeros((g,), jnp.float32))
        lat_ip = jnp.stack(cols, axis=1)
        for layer in range(4):
            wl = wl_ref[pl.ds(16 * layer, 16), :]
            latt_ref[pl.ds(g * layer, g), :] = jnp.dot(
                lat_ip, wl, preferred_element_type=jnp.float32)
        t1_ref[...] = jnp.dot(e_ref[...], wt_ref[...],
                              preferred_element_type=jnp.float32)
        t2_ref[...] = jnp.dot(t_ref[...], wb_ref[...],
                              preferred_element_type=jnp.float32)

    return pl.pallas_call(
        body,
        out_shape=(jax.ShapeDtypeStruct((4 * g, 128), jnp.float32),
                   jax.ShapeDtypeStruct((128, 128), jnp.float32),
                   jax.ShapeDtypeStruct((g, 128), jnp.float32)),
    )(lat9, wlat_all, emb_pad, wtop, t, wbot)


def _tc_h0(at3, n2g3, t1, t2, b, latt0, wi, wj, n):
    nb = n // _BN

    def body(at_ref, ng_ref, t1_ref, t2_ref, b_ref, lt_ref, wi_ref, wj_ref,
             h_ref, hs_ref, hd_ref):
        oh_at = _onehot_from(at_ref[0, 0, :], 128)
        oh_g = _onehot_from(ng_ref[0, 0, :], 64)
        h = (jnp.dot(oh_at, t1_ref[...], preferred_element_type=jnp.float32)
             + jnp.dot(oh_g, t2_ref[...], preferred_element_type=jnp.float32)
             + b_ref[...])
        h_ref[...] = h
        hs_ref[...] = (jnp.dot(h, wi_ref[...], preferred_element_type=jnp.float32)
                       + jnp.dot(oh_g, lt_ref[...], preferred_element_type=jnp.float32))
        hd_ref[...] = jnp.dot(h, wj_ref[...], preferred_element_type=jnp.float32)

    fixed = pl.BlockSpec(None, None)
    return pl.pallas_call(
        body,
        grid=(nb,),
        in_specs=[pl.BlockSpec((1, 1, _BN), lambda i: (i, 0, 0)),
                  pl.BlockSpec((1, 1, _BN), lambda i: (i, 0, 0)),
                  fixed, fixed, fixed, fixed, fixed, fixed],
        out_specs=[pl.BlockSpec((_BN, 128), lambda i: (i, 0)),
                   pl.BlockSpec((_BN, 128), lambda i: (i, 0)),
                   pl.BlockSpec((_BN, 128), lambda i: (i, 0))],
        out_shape=(jax.ShapeDtypeStruct((n, 128), jnp.float32),
                   jax.ShapeDtypeStruct((n, 128), jnp.float32),
                   jax.ShapeDtypeStruct((n, 128), jnp.float32)),
    )(at3, n2g3, t1, t2, b, latt0, wi, wj)


def _tc_dis(fdT, fmapT):
    e = fdT.shape[1]
    be = 6400
    nb = e // be

    def body(d_ref, f_ref, o_ref):
        ang_t = jnp.dot(f_ref[...], d_ref[...],
                        preferred_element_type=jnp.float32)
        row = lax.broadcasted_iota(jnp.int32, ang_t.shape, 0)
        dis_t = jnp.where(row < 32, jnp.sin(ang_t), jnp.cos(ang_t))
        o_ref[...] = dis_t.T

    fixed = pl.BlockSpec(None, None)
    return pl.pallas_call(
        body,
        grid=(nb,),
        in_specs=[pl.BlockSpec((4, be), lambda i: (0, i)),
                  fixed],
        out_specs=pl.BlockSpec((be, 64), lambda i: (i, 0)),
        out_shape=jax.ShapeDtypeStruct((e, 64), jnp.float32),
    )(fdT, fmapT)


def _tc_edge(gsd, dis, wdis, w2, b1, b2):
    e = gsd.shape[0]
    nb = e // _BE

    def body(s_ref, x_ref, wd_ref, w2_ref, b1_ref, b2_ref, o_ref):
        pre = (s_ref[...].astype(jnp.float32) + b1_ref[...]
               + jnp.dot(x_ref[...], wd_ref[...],
                         preferred_element_type=jnp.float32))
        s1 = _silu(pre)
        z = jnp.dot(s1, w2_ref[...], preferred_element_type=jnp.float32) + b2_ref[...]
        o_ref[...] = _silu(z)

    fixed = pl.BlockSpec(None, None)
    return pl.pallas_call(
        body,
        grid=(nb,),
        in_specs=[pl.BlockSpec((_BE, 128), lambda i: (i, 0)),
                  pl.BlockSpec((_BE, 64), lambda i: (i, 0)),
                  fixed, fixed, fixed, fixed],
        out_specs=pl.BlockSpec((_BE, 128), lambda i: (i, 0)),
        out_shape=jax.ShapeDtypeStruct((e, 128), jnp.float32),
    )(gsd, dis, wdis, w2, b1, b2)


def _tc_rdenom(cnt2):
    npd = cnt2.shape[1]
    bn = 632

    def body(c_ref, o_ref):
        c = c_ref[0, :, 0:1] + c_ref[1, :, 0:1]
        o_ref[...] = jnp.broadcast_to(1.0 / jnp.maximum(c, 1.0), (bn, 8))

    return pl.pallas_call(
        body,
        grid=(npd // bn,),
        in_specs=[pl.BlockSpec((2, bn, 128), lambda i: (0, i, 0))],
        out_specs=pl.BlockSpec((bn, 8), lambda i: (i, 0)),
        out_shape=jax.ShapeDtypeStruct((npd, 8), jnp.float32),
    )(cnt2)


def _tc_node(h, agg2, rd, w1h, w1a, b1, w2, b2, nxt):
    n = h.shape[0]
    nb = n // _BN

    def body(h_ref, a_ref, c_ref, w1h_ref, w1a_ref, b1_ref, w2_ref, b2_ref,
             *rest):
        h = h_ref[...]
        agg = (a_ref[0] + a_ref[1]) * c_ref[:, 0:1]
        z1 = _silu(jnp.dot(h, w1h_ref[...], preferred_element_type=jnp.float32)
                   + jnp.dot(agg, w1a_ref[...], preferred_element_type=jnp.float32)
                   + b1_ref[...])
        z2 = _silu(jnp.dot(z1, w2_ref[...], preferred_element_type=jnp.float32)
                   + b2_ref[...])
        hn = h + z2
        if nxt is None:
            (ho_ref,) = rest
            ho_ref[...] = hn
        else:
            ng_ref, lt_ref, wi_ref, wj_ref, ho_ref, hs_ref, hd_ref = rest
            ho_ref[...] = hn
            oh_g = _onehot_from(ng_ref[0, 0, :], 64)
            hs_ref[...] = (jnp.dot(hn, wi_ref[...], preferred_element_type=jnp.float32)
                           + jnp.dot(oh_g, lt_ref[...], preferred_element_type=jnp.float32))
            hd_ref[...] = jnp.dot(hn, wj_ref[...], preferred_element_type=jnp.float32)

    fixed = pl.BlockSpec(None, None)
    in_specs = [pl.BlockSpec((_BN, 128), lambda i: (i, 0)),
                pl.BlockSpec((2, _BN, 128), lambda i: (0, i, 0)),
                pl.BlockSpec((_BN, 8), lambda i: (i, 0)),
                fixed, fixed, fixed, fixed, fixed]
    args = [h, agg2, rd, w1h, w1a, b1, w2, b2]
    out_specs = [pl.BlockSpec((_BN, 128), lambda i: (i, 0))]
    out_shape = [jax.ShapeDtypeStruct((n, 128), jnp.float32)]
    if nxt is not None:
        n2g3, latt_l, wi, wj = nxt
        in_specs += [pl.BlockSpec((1, 1, _BN), lambda i: (i, 0, 0)),
                     fixed, fixed, fixed]
        args += [n2g3, latt_l, wi, wj]
        out_specs += [pl.BlockSpec((_BN, 128), lambda i: (i, 0)),
                      pl.BlockSpec((_BN, 128), lambda i: (i, 0))]
        out_shape += [jax.ShapeDtypeStruct((n, 128), jnp.float32),
                      jax.ShapeDtypeStruct((n, 128), jnp.float32)]
    res = pl.pallas_call(
        body,
        grid=(nb,),
        in_specs=in_specs,
        out_specs=out_specs,
        out_shape=out_shape,
    )(*args)
    return res if nxt is not None else (res[0],)


def _tc_pool(h, n2g3, coordw, n):
    nb = n // _BN

    def body(h_ref, ng_ref, cw_ref, co_ref, gs_ref, gc_ref):
        i = pl.program_id(0)
        h = h_ref[...]
        co_ref[...] = jnp.dot(h, cw_ref[...], preferred_element_type=jnp.float32)
        ids = ng_ref[0, 0, :]
        oht = (lax.broadcasted_iota(jnp.int32, (64, _BN), 0) == ids[None, :]
               ).astype(jnp.float32)

        @pl.when(i == 0)
        def _():
            gs_ref[...] = jnp.zeros_like(gs_ref)
            gc_ref[...] = jnp.zeros_like(gc_ref)

        gs_ref[...] += jnp.dot(oht, h, preferred_element_type=jnp.float32)
        gc_ref[...] += jnp.dot(oht, jnp.ones((_BN, 128), jnp.float32),
                               preferred_element_type=jnp.float32)

    fixed = pl.BlockSpec(None, None)
    return pl.pallas_call(
        body,
        grid=(nb,),
        in_specs=[pl.BlockSpec((_BN, 128), lambda i: (i, 0)),
                  pl.BlockSpec((1, 1, _BN), lambda i: (i, 0, 0)),
                  fixed],
        out_specs=[pl.BlockSpec((_BN, 8), lambda i: (i, 0)),
                   pl.BlockSpec((64, 128), lambda i: (0, 0)),
                   pl.BlockSpec((64, 128), lambda i: (0, 0))],
        out_shape=(jax.ShapeDtypeStruct((n, 8), jnp.float32),
                   jax.ShapeDtypeStruct((64, 128), jnp.float32),
                   jax.ShapeDtypeStruct((64, 128), jnp.float32)),
    )(h, n2g3, coordw)


def _tc_lattice(gsum, gcnt, latw, lat9):
    g = lat9.shape[0]

    def body(gs_ref, gc_ref, w_ref, l_ref, o_ref):
        gf = gs_ref[...] / jnp.maximum(gc_ref[...], 1.0)
        gl = jnp.dot(gf, w_ref[...], preferred_element_type=jnp.float32)
        l = l_ref[...]
        cols = []
        for i in range(3):
            for kk in range(3):
                acc = gl[:, 3 * i + 0] * l[:, 0 + kk]
                acc = acc + gl[:, 3 * i + 1] * l[:, 3 + kk]
                acc = acc + gl[:, 3 * i + 2] * l[:, 6 + kk]
                cols.append(acc)
        for _ in range(7):
            cols.append(jnp.zeros((g,), jnp.float32))
        o_ref[...] = jnp.stack(cols, axis=1)

    return pl.pallas_call(
        body,
        out_shape=jax.ShapeDtypeStruct((g, 16), jnp.float32),
    )(gsum, gcnt, latw, lat9)


def kernel(atom_types, frac_coords, lattices, edge_index, node2graph, t, params):
    n = atom_types.shape[0]
    e = edge_index.shape[1]
    g = lattices.shape[0]
    hid = 128

    src = edge_index[0].astype(jnp.int32).reshape(1, e)
    dst = edge_index[1].astype(jnp.int32).reshape(1, e)
    at3 = atom_types.astype(jnp.int32).reshape(n // _BN, 1, _BN)
    n2g3 = node2graph.astype(jnp.int32).reshape(n // _BN, 1, _BN)

    p = params
    layers = p["layers"]

    wtop = p["atom_latent_W"][:hid]
    wbot = p["atom_latent_W"][hid:]
    b_al = p["atom_latent_b"].reshape(1, hid)
    emb_pad = jnp.zeros((128, hid), jnp.float32).at[:p["node_emb"].shape[0]].set(
        p["node_emb"])
    lat9 = jnp.pad(lattices.reshape(g, 9), ((0, 0), (0, 7)))
    wlat_all = jnp.concatenate(
        [jnp.pad(lp["eW1"][2 * hid:2 * hid + 9], ((0, 7), (0, 0)))
         for lp in layers], axis=0)
    wi = [lp["eW1"][:hid] for lp in layers]
    wj = [lp["eW1"][hid:2 * hid] for lp in layers]
    wdis = []
    for lp in layers:
        wd = lp["eW1"][2 * hid + 9:]
        wdis.append(jnp.concatenate([
            wd[:30], jnp.zeros((2, hid), jnp.float32),
            wd[30:], jnp.zeros((2, hid), jnp.float32)], axis=0))
    eb1 = [lp["eb1"].reshape(1, hid) for lp in layers]
    ew2 = [lp["eW2"] for lp in layers]
    eb2 = [lp["eb2"].reshape(1, hid) for lp in layers]
    nw1h = [lp["nW1"][:hid] for lp in layers]
    nw1a = [lp["nW1"][hid:] for lp in layers]
    nb1 = [lp["nb1"].reshape(1, hid) for lp in layers]
    nw2 = [lp["nW2"] for lp in layers]
    nb2 = [lp["nb2"].reshape(1, hid) for lp in layers]
    coordw = jnp.pad(p["coord_W"], ((0, 0), (0, 5)))
    latw = jnp.pad(p["lattice_W"], ((0, 0), (0, 7)))

    freqs = 2.0 * math.pi * np.arange(NFREQ, dtype=np.float32)
    fmap_np = np.zeros((64, 4), np.float32)
    for j in range(3):
        for f in range(NFREQ):
            fmap_np[j * NFREQ + f, j] = freqs[f]
            fmap_np[32 + j * NFREQ + f, j] = freqs[f]
    fmap_t = jnp.asarray(fmap_np)

    frac_flat = jnp.pad(frac_coords, ((0, 0), (0, 1))).reshape(-1)
    npad = ((n + 8 * _SC_TILES - 1) // (8 * _SC_TILES)) * (8 * _SC_TILES)
    zeros_2nd = jnp.zeros((_SC_CORES, npad, hid), jnp.float32)
    zeros_nc = jnp.zeros((npad, hid), jnp.float32)
    ones_w = jnp.ones((_SC_W, hid), jnp.float32)

    eh = e // 2
    src1 = lax.slice(src, (0, 0), (1, eh))
    src2 = lax.slice(src, (0, eh), (1, e))
    dst1 = lax.slice(dst, (0, 0), (1, eh))
    dst2 = lax.slice(dst, (0, eh), (1, e))

    latt_all, t1, t2 = _tc_tables(lat9, wlat_all, emb_pad, wtop, t, wbot)
    cnt2 = _sc_counts(src, ones_w, zeros_nc)
    rd = _tc_rdenom(cnt2)
    fdT1 = _sc_fdiff(frac_flat, src1, dst1)
    fdT2 = _sc_fdiff(frac_flat, src2, dst2)
    dis1 = _tc_dis(fdT1, fmap_t)
    dis2 = _tc_dis(fdT2, fmap_t)
    h, hs, hd = _tc_h0(at3, n2g3, t1, t2, b_al,
                       lax.slice_in_dim(latt_all, 0, g), wi[0], wj[0], n)

    for l in range(4):
        gsd1 = _sc_gather_add(hs, src1, hd, dst1)
        ef1 = _tc_edge(gsd1, dis1, wdis[l], ew2[l], eb1[l], eb2[l])
        gsd2 = _sc_gather_add(hs, src2, hd, dst2)
        agg2a = _sc_scatter_rows(ef1, src1, zeros_2nd)
        ef2 = _tc_edge(gsd2, dis2, wdis[l], ew2[l], eb1[l], eb2[l])
        agg2 = _sc_scatter_rows(ef2, src2, agg2a)
        if l < 3:
            nxt = (n2g3, lax.slice_in_dim(latt_all, (l + 1) * g, (l + 2) * g),
                   wi[l + 1], wj[l + 1])
            h, hs, hd = _tc_node(h, agg2, rd, nw1h[l], nw1a[l], nb1[l],
                                 nw2[l], nb2[l], nxt)
        else:
            (h,) = _tc_node(h, agg2, rd, nw1h[l], nw1a[l], nb1[l],
                            nw2[l], nb2[l], None)

    coord8, gsum, gcnt = _tc_pool(h, n2g3, coordw, n)
    lo16 = _tc_lattice(gsum, gcnt, latw, lat9)

    coord_out = coord8[:, :3]
    lattice_out = lo16[:, :9].reshape(g, 3, 3)
    return lattice_out, coord_out


# speedup vs baseline: 13.0681x; 1.0969x over previous
import dataclasses
import functools
import math

import jax
import jax.numpy as jnp
import numpy as np
from jax import lax
from jax.experimental import pallas as pl
from jax.experimental.pallas import tpu as pltpu
from jax.experimental.pallas import tpu_sc as plsc

NFREQ = 10


_SC_CORES = 2
_SC_TILES = 16
_SC_W = 128


def _sc_mesh():
    return plsc.VectorSubcoreMesh(
        core_axis_name="c", subcore_axis_name="s",
        num_cores=_SC_CORES, num_subcores=_SC_TILES)


def _sc_gather_add(t1, i1, t2, i2):
    n, d = t1.shape
    e = i1.shape[1]
    w = _SC_W

    @functools.partial(
        pl.kernel,
        out_type=jax.ShapeDtypeStruct((e, d), jnp.float32),
        mesh=_sc_mesh(),
        scratch_types=[pltpu.VMEM_SHARED((n, d), jnp.float32)])
    def k(t1_hbm, i1_hbm, t2_hbm, i2_hbm, o_hbm, t1s):
        sid = lax.axis_index("s")

        @pl.when(sid == 0)
        def _():
            pltpu.sync_copy(t1_hbm, t1s)

        plsc.subcore_barrier()

        def body(i1_v, i2_v, o_v):
            pltpu.sync_copy(t1s.at[i1_v.at[0]], o_v)
            pltpu.sync_copy(t2_hbm.at[i2_v.at[0]], o_v, add=True)

        pltpu.emit_pipeline(
            body,
            grid=(e // w,),
            in_specs=[pl.BlockSpec((1, w), lambda i: (0, i)),
                      pl.BlockSpec((1, w), lambda i: (0, i))],
            out_specs=[pl.BlockSpec((w, d), lambda i: (i, 0))],
            core_axis_name=("c", "s"),
            dimension_semantics=(pltpu.PARALLEL,),
        )(i1_hbm, i2_hbm, o_hbm)

    return k(t1, i1, t2, i2)


def _sc_fdiff(frac_flat, src, dst):
    e = src.shape[1]
    w = _SC_W
    nflat = frac_flat.shape[0]

    cp = pltpu.CompilerParams()
    if "needs_layout_passes" in pltpu.CompilerParams.__dataclass_fields__:
        cp = dataclasses.replace(cp, needs_layout_passes=False)

    @functools.partial(
        pl.kernel,
        out_type=jax.ShapeDtypeStruct((4, e), jnp.float32),
        mesh=_sc_mesh(),
        compiler_params=cp,
        scratch_types=[pltpu.VMEM((nflat,), jnp.float32)])
    def k(f_hbm, s_hbm, d_hbm, o_hbm, tbl):
        pltpu.sync_copy(f_hbm, tbl)

        def body(s_v, d_v, o_v):
            for gi in range(w // 16):
                sl = pl.ds(gi * 16, 16)
                s16 = s_v[0, sl] * 4
                d16 = d_v[0, sl] * 4
                for c in range(3):
                    fs = plsc.load_gather(tbl, [s16 + c])
                    fd = plsc.load_gather(tbl, [d16 + c])
                    o_v[c, sl] = fd - fs
                o_v[3, sl] = jnp.zeros((16,), jnp.float32)

        pltpu.emit_pipeline(
            body,
            grid=(e // w,),
            in_specs=[pl.BlockSpec((1, w), lambda i: (0, i)),
                      pl.BlockSpec((1, w), lambda i: (0, i))],
            out_specs=[pl.BlockSpec((4, w), lambda i: (0, i))],
            core_axis_name=("c", "s"),
            dimension_semantics=(pltpu.PARALLEL,),
        )(s_hbm, d_hbm, o_hbm)

    return k(frac_flat, src, dst)


def _sc_scatter_rows(vals, idx, init):
    e, d = vals.shape
    n = init.shape[1]
    w = _SC_W
    rows = n // _SC_TILES

    @functools.partial(
        pl.kernel,
        out_type=jax.ShapeDtypeStruct((_SC_CORES, n, d), jnp.float32),
        mesh=_sc_mesh(),
        scratch_types=[pltpu.VMEM_SHARED((n, d), jnp.float32)])
    def k(v_hbm, i_hbm, z_hbm, o_hbm, acc):
        cid = lax.axis_index("c")
        sid = lax.axis_index("s")
        pltpu.sync_copy(z_hbm.at[cid].at[pl.ds(sid * rows, rows)],
                        acc.at[pl.ds(sid * rows, rows)])
        plsc.subcore_barrier()

        def body(v_v, i_v):
            pltpu.sync_copy(v_v, acc.at[i_v.at[0]], add=True)

        pltpu.emit_pipeline(
            body,
            grid=(e // w,),
            in_specs=[pl.BlockSpec((w, d), lambda i: (i, 0)),
                      pl.BlockSpec((1, w), lambda i: (0, i))],
            out_specs=[],
            core_axis_name=("c", "s"),
            dimension_semantics=(pltpu.PARALLEL,),
        )(v_hbm, i_hbm)

        plsc.subcore_barrier()
        pltpu.sync_copy(acc.at[pl.ds(sid * rows, rows)],
                        o_hbm.at[cid].at[pl.ds(sid * rows, rows)])

    return k(vals, idx, init)


def _sc_counts(idx, ones, zeros):
    e = idx.shape[1]
    n, dc = zeros.shape
    w = _SC_W
    rows = n // _SC_TILES

    @functools.partial(
        pl.kernel,
        out_type=jax.ShapeDtypeStruct((_SC_CORES, n, dc), jnp.float32),
        mesh=_sc_mesh(),
        scratch_types=[pltpu.VMEM((w, dc), jnp.float32),
                       pltpu.VMEM_SHARED((n, dc), jnp.float32)])
    def k(i_hbm, one_hbm, z_hbm, o_hbm, ones_v, acc):
        cid = lax.axis_index("c")
        sid = lax.axis_index("s")
        pltpu.sync_copy(one_hbm, ones_v)
        pltpu.sync_copy(z_hbm.at[pl.ds(sid * rows, rows)],
                        acc.at[pl.ds(sid * rows, rows)])
        plsc.subcore_barrier()

        def body(i_v):
            pltpu.sync_copy(ones_v, acc.at[i_v.at[0]], add=True)

        pltpu.emit_pipeline(
            body,
            grid=(e // w,),
            in_specs=[pl.BlockSpec((1, w), lambda i: (0, i))],
            out_specs=[],
            core_axis_name=("c", "s"),
            dimension_semantics=(pltpu.PARALLEL,),
        )(i_hbm)

        plsc.subcore_barrier()
        pltpu.sync_copy(acc.at[pl.ds(sid * rows, rows)],
                        o_hbm.at[cid].at[pl.ds(sid * rows, rows)])

    return k(idx, ones, zeros)


_BN = 1000
_BE = 4000


def _silu(x):
    return x * jax.nn.sigmoid(x)


def _onehot_from(ids, nclass):
    return (ids[:, None] == lax.broadcasted_iota(jnp.int32, (ids.shape[0], nclass), 1)
            ).astype(jnp.float32)


def _tc_tables(lat9, wlat_all, emb_pad, wtop, t, wbot):
    g = lat9.shape[0]

    def body(l_ref, wl_ref, e_ref, wt_ref, t_ref, wb_ref,
             latt_ref, t1_ref, t2_ref):
        l = l_ref[...]
        cols = []
        for i in range(3):
            for kk in range(3):
                acc = l[:, 3 * i + 0] * l[:, 3 * kk + 0]
                acc = acc + l[:, 3 * i + 1] * l[:, 3 * kk + 1]
                acc = acc + l[:, 3 * i + 2] * l[:, 3 * kk + 2]
                cols.append(acc)
        for _ in range(7):
            cols.append(jnp.zeros((g,), jnp.float32))
        lat_ip = jnp.stack(cols, axis=1)
        for layer in range(4):
            wl = wl_ref[pl.ds(16 * layer, 16), :]
            latt_ref[pl.ds(g * layer, g), :] = jnp.dot(
                lat_ip, wl, preferred_element_type=jnp.float32)
        t1_ref[...] = jnp.dot(e_ref[...], wt_ref[...],
                              preferred_element_type=jnp.float32)
        t2_ref[...] = jnp.dot(t_ref[...], wb_ref[...],
                              preferred_element_type=jnp.float32)

    return pl.pallas_call(
        body,
        out_shape=(jax.ShapeDtypeStruct((4 * g, 128), jnp.float32),
                   jax.ShapeDtypeStruct((128, 128), jnp.float32),
                   jax.ShapeDtypeStruct((g, 128), jnp.float32)),
    )(lat9, wlat_all, emb_pad, wtop, t, wbot)


def _tc_h0(at3, n2g3, t1, t2, b, latt0, wi, wj, n):
    nb = n // _BN

    def body(at_ref, ng_ref, t1_ref, t2_ref, b_ref, lt_ref, wi_ref, wj_ref,
             h_ref, hs_ref, hd_ref):
        oh_at = _onehot_from(at_ref[0, 0, :], 128)
        oh_g = _onehot_from(ng_ref[0, 0, :], 64)
        h = (jnp.dot(oh_at, t1_ref[...], preferred_element_type=jnp.float32)
             + jnp.dot(oh_g, t2_ref[...], preferred_element_type=jnp.float32)
             + b_ref[...])
        h_ref[...] = h
        hs_ref[...] = (jnp.dot(h, wi_ref[...], preferred_element_type=jnp.float32)
                       + jnp.dot(oh_g, lt_ref[...], preferred_element_type=jnp.float32))
        hd_ref[...] = jnp.dot(h, wj_ref[...], preferred_element_type=jnp.float32)

    fixed = pl.BlockSpec(None, None)
    return pl.pallas_call(
        body,
        grid=(nb,),
        in_specs=[pl.BlockSpec((1, 1, _BN), lambda i: (i, 0, 0)),
                  pl.BlockSpec((1, 1, _BN), lambda i: (i, 0, 0)),
                  fixed, fixed, fixed, fixed, fixed, fixed],
        out_specs=[pl.BlockSpec((_BN, 128), lambda i: (i, 0)),
                   pl.BlockSpec((_BN, 128), lambda i: (i, 0)),
                   pl.BlockSpec((_BN, 128), lambda i: (i, 0))],
        out_shape=(jax.ShapeDtypeStruct((n, 128), jnp.float32),
                   jax.ShapeDtypeStruct((n, 128), jnp.float32),
                   jax.ShapeDtypeStruct((n, 128), jnp.float32)),
    )(at3, n2g3, t1, t2, b, latt0, wi, wj)


def _tc_dis(fdT, fmapT):
    e = fdT.shape[1]
    be = 6400
    nb = e // be

    def body(d_ref, f_ref, o_ref):
        ang_t = jnp.dot(f_ref[...], d_ref[...],
                        preferred_element_type=jnp.float32)
        row = lax.broadcasted_iota(jnp.int32, ang_t.shape, 0)
        dis_t = jnp.where(row < 32, jnp.sin(ang_t), jnp.cos(ang_t))
        o_ref[...] = dis_t.T

    fixed = pl.BlockSpec(None, None)
    return pl.pallas_call(
        body,
        grid=(nb,),
        in_specs=[pl.BlockSpec((4, be), lambda i: (0, i)),
                  fixed],
        out_specs=pl.BlockSpec((be, 64), lambda i: (i, 0)),
        out_shape=jax.ShapeDtypeStruct((e, 64), jnp.float32),
    )(fdT, fmapT)


def _tc_edge(gsd, dis, wdis, w2, b1, b2):
    e = gsd.shape[0]
    nb = e // _BE

    def body(s_ref, x_ref, wd_ref, w2_ref, b1_ref, b2_ref, o_ref):
        pre = (s_ref[...].astype(jnp.float32) + b1_ref[...]
               + jnp.dot(x_ref[...], wd_ref[...],
                         preferred_element_type=jnp.float32))
        s1 = _silu(pre)
        z = jnp.dot(s1, w2_ref[...], preferred_element_type=jnp.float32) + b2_ref[...]
        o_ref[...] = _silu(z)

    fixed = pl.BlockSpec(None, None)
    return pl.pallas_call(
        body,
        grid=(nb,),
        in_specs=[pl.BlockSpec((_BE, 128), lambda i: (i, 0)),
                  pl.BlockSpec((_BE, 64), lambda i: (i, 0)),
                  fixed, fixed, fixed, fixed],
        out_specs=pl.BlockSpec((_BE, 128), lambda i: (i, 0)),
        out_shape=jax.ShapeDtypeStruct((e, 128), jnp.float32),
    )(gsd, dis, wdis, w2, b1, b2)


def _tc_rdenom(cnt2):
    npd = cnt2.shape[1]
    bn = 632

    def body(c_ref, o_ref):
        c = c_ref[0, :, 0:1] + c_ref[1, :, 0:1]
        o_ref[...] = jnp.broadcast_to(1.0 / jnp.maximum(c, 1.0), (bn, 8))

    return pl.pallas_call(
        body,
        grid=(npd // bn,),
        in_specs=[pl.BlockSpec((2, bn, 128), lambda i: (0, i, 0))],
        out_specs=pl.BlockSpec((bn, 8), lambda i: (i, 0)),
        out_shape=jax.ShapeDtypeStruct((npd, 8), jnp.float32),
    )(cnt2)


def _tc_node(h, agg2, rd, w1h, w1a, b1, w2, b2, nxt):
    n = h.shape[0]
    nb = n // _BN

    def body(h_ref, a_ref, c_ref, w1h_ref, w1a_ref, b1_ref, w2_ref, b2_ref,
             *rest):
        h = h_ref[...]
        agg = (a_ref[0] + a_ref[1]) * c_ref[:, 0:1]
        z1 = _silu(jnp.dot(h, w1h_ref[...], preferred_element_type=jnp.float32)
                   + jnp.dot(agg, w1a_ref[...], preferred_element_type=jnp.float32)
                   + b1_ref[...])
        z2 = _silu(jnp.dot(z1, w2_ref[...], preferred_element_type=jnp.float32)
                   + b2_ref[...])
        hn = h + z2
        if nxt is None:
            (ho_ref,) = rest
            ho_ref[...] = hn
        else:
            ng_ref, lt_ref, wi_ref, wj_ref, ho_ref, hs_ref, hd_ref = rest
            ho_ref[...] = hn
            oh_g = _onehot_from(ng_ref[0, 0, :], 64)
            hs_ref[...] = (jnp.dot(hn, wi_ref[...], preferred_element_type=jnp.float32)
                           + jnp.dot(oh_g, lt_ref[...], preferred_element_type=jnp.float32))
            hd_ref[...] = jnp.dot(hn, wj_ref[...], preferred_element_type=jnp.float32)

    fixed = pl.BlockSpec(None, None)
    in_specs = [pl.BlockSpec((_BN, 128), lambda i: (i, 0)),
                pl.BlockSpec((2, _BN, 128), lambda i: (0, i, 0)),
                pl.BlockSpec((_BN, 8), lambda i: (i, 0)),
                fixed, fixed, fixed, fixed, fixed]
    args = [h, agg2, rd, w1h, w1a, b1, w2, b2]
    out_specs = [pl.BlockSpec((_BN, 128), lambda i: (i, 0))]
    out_shape = [jax.ShapeDtypeStruct((n, 128), jnp.float32)]
    if nxt is not None:
        n2g3, latt_l, wi, wj = nxt
        in_specs += [pl.BlockSpec((1, 1, _BN), lambda i: (i, 0, 0)),
                     fixed, fixed, fixed]
        args += [n2g3, latt_l, wi, wj]
        out_specs += [pl.BlockSpec((_BN, 128), lambda i: (i, 0)),
                      pl.BlockSpec((_BN, 128), lambda i: (i, 0))]
        out_shape += [jax.ShapeDtypeStruct((n, 128), jnp.float32),
                      jax.ShapeDtypeStruct((n, 128), jnp.float32)]
    res = pl.pallas_call(
        body,
        grid=(nb,),
        in_specs=in_specs,
        out_specs=out_specs,
        out_shape=out_shape,
    )(*args)
    return res if nxt is not None else (res[0],)


def _tc_pool(h, n2g3, coordw, n):
    nb = n // _BN

    def body(h_ref, ng_ref, cw_ref, co_ref, gs_ref, gc_ref):
        i = pl.program_id(0)
        h = h_ref[...]
        co_ref[...] = jnp.dot(h, cw_ref[...], preferred_element_type=jnp.float32)
        ids = ng_ref[0, 0, :]
        oht = (lax.broadcasted_iota(jnp.int32, (64, _BN), 0) == ids[None, :]
               ).astype(jnp.float32)

        @pl.when(i == 0)
        def _():
            gs_ref[...] = jnp.zeros_like(gs_ref)
            gc_ref[...] = jnp.zeros_like(gc_ref)

        gs_ref[...] += jnp.dot(oht, h, preferred_element_type=jnp.float32)
        gc_ref[...] += jnp.dot(oht, jnp.ones((_BN, 128), jnp.float32),
                               preferred_element_type=jnp.float32)

    fixed = pl.BlockSpec(None, None)
    return pl.pallas_call(
        body,
        grid=(nb,),
        in_specs=[pl.BlockSpec((_BN, 128), lambda i: (i, 0)),
                  pl.BlockSpec((1, 1, _BN), lambda i: (i, 0, 0)),
                  fixed],
        out_specs=[pl.BlockSpec((_BN, 8), lambda i: (i, 0)),
                   pl.BlockSpec((64, 128), lambda i: (0, 0)),
                   pl.BlockSpec((64, 128), lambda i: (0, 0))],
        out_shape=(jax.ShapeDtypeStruct((n, 8), jnp.float32),
                   jax.ShapeDtypeStruct((64, 128), jnp.float32),
                   jax.ShapeDtypeStruct((64, 128), jnp.float32)),
    )(h, n2g3, coordw)


def _tc_lattice(gsum, gcnt, latw, lat9):
    g = lat9.shape[0]

    def body(gs_ref, gc_ref, w_ref, l_ref, o_ref):
        gf = gs_ref[...] / jnp.maximum(gc_ref[...], 1.0)
        gl = jnp.dot(gf, w_ref[...], preferred_element_type=jnp.float32)
        l = l_ref[...]
        cols = []
        for i in range(3):
            for kk in range(3):
                acc = gl[:, 3 * i + 0] * l[:, 0 + kk]
                acc = acc + gl[:, 3 * i + 1] * l[:, 3 + kk]
                acc = acc + gl[:, 3 * i + 2] * l[:, 6 + kk]
                cols.append(acc)
        for _ in range(7):
            cols.append(jnp.zeros((g,), jnp.float32))
        o_ref[...] = jnp.stack(cols, axis=1)

    return pl.pallas_call(
        body,
        out_shape=jax.ShapeDtypeStruct((g, 16), jnp.float32),
    )(gsum, gcnt, latw, lat9)


def kernel(atom_types, frac_coords, lattices, edge_index, node2graph, t, params):
    n = atom_types.shape[0]
    e = edge_index.shape[1]
    g = lattices.shape[0]
    hid = 128

    src = edge_index[0].astype(jnp.int32).reshape(1, e)
    dst = edge_index[1].astype(jnp.int32).reshape(1, e)
    at3 = atom_types.astype(jnp.int32).reshape(n // _BN, 1, _BN)
    n2g3 = node2graph.astype(jnp.int32).reshape(n // _BN, 1, _BN)

    p = params
    layers = p["layers"]

    wtop = p["atom_latent_W"][:hid]
    wbot = p["atom_latent_W"][hid:]
    b_al = p["atom_latent_b"].reshape(1, hid)
    emb_pad = jnp.zeros((128, hid), jnp.float32).at[:p["node_emb"].shape[0]].set(
        p["node_emb"])
    lat9 = jnp.pad(lattices.reshape(g, 9), ((0, 0), (0, 7)))
    wlat_all = jnp.concatenate(
        [jnp.pad(lp["eW1"][2 * hid:2 * hid + 9], ((0, 7), (0, 0)))
         for lp in layers], axis=0)
    wi = [lp["eW1"][:hid] for lp in layers]
    wj = [lp["eW1"][hid:2 * hid] for lp in layers]
    wdis = []
    for lp in layers:
        wd = lp["eW1"][2 * hid + 9:]
        wdis.append(jnp.concatenate([
            wd[:30], jnp.zeros((2, hid), jnp.float32),
            wd[30:], jnp.zeros((2, hid), jnp.float32)], axis=0))
    eb1 = [lp["eb1"].reshape(1, hid) for lp in layers]
    ew2 = [lp["eW2"] for lp in layers]
    eb2 = [lp["eb2"].reshape(1, hid) for lp in layers]
    nw1h = [lp["nW1"][:hid] for lp in layers]
    nw1a = [lp["nW1"][hid:] for lp in layers]
    nb1 = [lp["nb1"].reshape(1, hid) for lp in layers]
    nw2 = [lp["nW2"] for lp in layers]
    nb2 = [lp["nb2"].reshape(1, hid) for lp in layers]
    coordw = jnp.pad(p["coord_W"], ((0, 0), (0, 5)))
    latw = jnp.pad(p["lattice_W"], ((0, 0), (0, 7)))

    freqs = 2.0 * math.pi * np.arange(NFREQ, dtype=np.float32)
    fmap_np = np.zeros((64, 4), np.float32)
    for j in range(3):
        for f in range(NFREQ):
            fmap_np[j * NFREQ + f, j] = freqs[f]
            fmap_np[32 + j * NFREQ + f, j] = freqs[f]
    fmap_t = jnp.asarray(fmap_np)

    frac_flat = jnp.pad(frac_coords, ((0, 0), (0, 1))).reshape(-1)
    npad = ((n + 8 * _SC_TILES - 1) // (8 * _SC_TILES)) * (8 * _SC_TILES)
    zeros_2nd = jnp.zeros((_SC_CORES, npad, hid), jnp.float32)
    zeros_nc = jnp.zeros((npad, hid), jnp.float32)
    ones_w = jnp.ones((_SC_W, hid), jnp.float32)

    eh = e // 2
    src1 = lax.slice(src, (0, 0), (1, eh))
    src2 = lax.slice(src, (0, eh), (1, e))
    dst1 = lax.slice(dst, (0, 0), (1, eh))
    dst2 = lax.slice(dst, (0, eh), (1, e))

    latt_all, t1, t2 = _tc_tables(lat9, wlat_all, emb_pad, wtop, t, wbot)
    cnt2 = _sc_counts(src, ones_w, zeros_nc)
    rd = _tc_rdenom(cnt2)
    fdT1 = _sc_fdiff(frac_flat, src1, dst1)
    fdT2 = _sc_fdiff(frac_flat, src2, dst2)
    dis1 = _tc_dis(fdT1, fmap_t)
    dis2 = _tc_dis(fdT2, fmap_t)
    h, hs, hd = _tc_h0(at3, n2g3, t1, t2, b_al,
                       lax.slice_in_dim(latt_all, 0, g), wi[0], wj[0], n)

    for l in range(4):
        gsd1 = _sc_gather_add(hs, src1, hd, dst1)
        ef1 = _tc_edge(gsd1, dis1, wdis[l], ew2[l], eb1[l], eb2[l])
        gsd2 = _sc_gather_add(hs, src2, hd, dst2)
        agg2a = _sc_scatter_rows(ef1, src1, zeros_2nd)
        ef2 = _tc_edge(gsd2, dis2, wdis[l], ew2[l], eb1[l], eb2[l])
        agg2 = _sc_scatter_rows(ef2, src2, agg2a)
        if l < 3:
            nxt = (n2g3, lax.slice_in_dim(latt_all, (l + 1) * g, (l + 2) * g),
                   wi[l + 1], wj[l + 1])
            h, hs, hd = _tc_node(h, agg2, rd, nw1h[l], nw1a[l], nb1[l],
                                 nw2[l], nb2[l], nxt)
        else:
            (h,) = _tc_node(h, agg2, rd, nw1h[l], nw1a[l], nb1[l],
                            nw2[l], nb2[l], None)

    coord8, gsum, gcnt = _tc_pool(h, n2g3, coordw, n)
    lo16 = _tc_lattice(gsum, gcnt, latw, lat9)

    coord_out = coord8[:, :3]
    lattice_out = lo16[:, :9].reshape(g, 3, 3)
    return lattice_out, coord_out


# speedup vs baseline: 13.6323x; 1.0432x over previous
import dataclasses
import functools
import math

import jax
import jax.numpy as jnp
import numpy as np
from jax import lax
from jax.experimental import pallas as pl
from jax.experimental.pallas import tpu as pltpu
from jax.experimental.pallas import tpu_sc as plsc

NFREQ = 10


_SC_CORES = 2
_SC_TILES = 16
_SC_W = 128


def _sc_mesh():
    return plsc.VectorSubcoreMesh(
        core_axis_name="c", subcore_axis_name="s",
        num_cores=_SC_CORES, num_subcores=_SC_TILES)


def _sc_gather_add(t1, i1, t2, i2):
    n, d = t1.shape
    e = i1.shape[1]
    w = _SC_W

    @functools.partial(
        pl.kernel,
        out_type=jax.ShapeDtypeStruct((e, d), jnp.float32),
        mesh=_sc_mesh(),
        scratch_types=[pltpu.VMEM_SHARED((n, d), jnp.float32)])
    def k(t1_hbm, i1_hbm, t2_hbm, i2_hbm, o_hbm, t1s):
        sid = lax.axis_index("s")

        @pl.when(sid == 0)
        def _():
            pltpu.sync_copy(t1_hbm, t1s)

        plsc.subcore_barrier()

        def body(i1_v, i2_v, o_v):
            pltpu.sync_copy(t1s.at[i1_v.at[0]], o_v)
            pltpu.sync_copy(t2_hbm.at[i2_v.at[0]], o_v, add=True)

        pltpu.emit_pipeline(
            body,
            grid=(e // w,),
            in_specs=[pl.BlockSpec((1, w), lambda i: (0, i)),
                      pl.BlockSpec((1, w), lambda i: (0, i))],
            out_specs=[pl.BlockSpec((w, d), lambda i: (i, 0))],
            core_axis_name=("c", "s"),
            dimension_semantics=(pltpu.PARALLEL,),
        )(i1_hbm, i2_hbm, o_hbm)

    return k(t1, i1, t2, i2)


def _sc_fdiff(frac_flat, src, dst):
    e = src.shape[1]
    w = _SC_W
    nflat = frac_flat.shape[0]

    cp = pltpu.CompilerParams()
    if "needs_layout_passes" in pltpu.CompilerParams.__dataclass_fields__:
        cp = dataclasses.replace(cp, needs_layout_passes=False)

    @functools.partial(
        pl.kernel,
        out_type=jax.ShapeDtypeStruct((4, e), jnp.float32),
        mesh=_sc_mesh(),
        compiler_params=cp,
        scratch_types=[pltpu.VMEM((nflat,), jnp.float32)])
    def k(f_hbm, s_hbm, d_hbm, o_hbm, tbl):
        pltpu.sync_copy(f_hbm, tbl)

        def body(s_v, d_v, o_v):
            for gi in range(w // 16):
                sl = pl.ds(gi * 16, 16)
                s16 = s_v[0, sl] * 4
                d16 = d_v[0, sl] * 4
                for c in range(3):
                    fs = plsc.load_gather(tbl, [s16 + c])
                    fd = plsc.load_gather(tbl, [d16 + c])
                    o_v[c, sl] = fd - fs
                o_v[3, sl] = jnp.zeros((16,), jnp.float32)

        pltpu.emit_pipeline(
            body,
            grid=(e // w,),
            in_specs=[pl.BlockSpec((1, w), lambda i: (0, i)),
                      pl.BlockSpec((1, w), lambda i: (0, i))],
            out_specs=[pl.BlockSpec((4, w), lambda i: (0, i))],
            core_axis_name=("c", "s"),
            dimension_semantics=(pltpu.PARALLEL,),
        )(s_hbm, d_hbm, o_hbm)

    return k(frac_flat, src, dst)


def _sc_scatter_rows(vals, idx, init):
    e, d = vals.shape
    n = init.shape[1]
    w = _SC_W
    rows = n // _SC_TILES

    @functools.partial(
        pl.kernel,
        out_type=jax.ShapeDtypeStruct((_SC_CORES, n, d), jnp.float32),
        mesh=_sc_mesh(),
        scratch_types=[pltpu.VMEM_SHARED((n, d), jnp.float32)])
    def k(v_hbm, i_hbm, z_hbm, o_hbm, acc):
        cid = lax.axis_index("c")
        sid = lax.axis_index("s")
        pltpu.sync_copy(z_hbm.at[cid].at[pl.ds(sid * rows, rows)],
                        acc.at[pl.ds(sid * rows, rows)])
        plsc.subcore_barrier()

        def body(v_v, i_v):
            pltpu.sync_copy(v_v, acc.at[i_v.at[0]], add=True)

        pltpu.emit_pipeline(
            body,
            grid=(e // w,),
            in_specs=[pl.BlockSpec((w, d), lambda i: (i, 0)),
                      pl.BlockSpec((1, w), lambda i: (0, i))],
            out_specs=[],
            core_axis_name=("c", "s"),
            dimension_semantics=(pltpu.PARALLEL,),
        )(v_hbm, i_hbm)

        plsc.subcore_barrier()
        pltpu.sync_copy(acc.at[pl.ds(sid * rows, rows)],
                        o_hbm.at[cid].at[pl.ds(sid * rows, rows)])

    return k(vals, idx, init)


def _sc_counts(idx, ones, zeros):
    e = idx.shape[1]
    n, dc = zeros.shape
    w = _SC_W
    rows = n // _SC_TILES

    @functools.partial(
        pl.kernel,
        out_type=jax.ShapeDtypeStruct((_SC_CORES, n, dc), jnp.float32),
        mesh=_sc_mesh(),
        scratch_types=[pltpu.VMEM((w, dc), jnp.float32),
                       pltpu.VMEM_SHARED((n, dc), jnp.float32)])
    def k(i_hbm, one_hbm, z_hbm, o_hbm, ones_v, acc):
        cid = lax.axis_index("c")
        sid = lax.axis_index("s")
        pltpu.sync_copy(one_hbm, ones_v)
        pltpu.sync_copy(z_hbm.at[pl.ds(sid * rows, rows)],
                        acc.at[pl.ds(sid * rows, rows)])
        plsc.subcore_barrier()

        def body(i_v):
            pltpu.sync_copy(ones_v, acc.at[i_v.at[0]], add=True)

        pltpu.emit_pipeline(
            body,
            grid=(e // w,),
            in_specs=[pl.BlockSpec((1, w), lambda i: (0, i))],
            out_specs=[],
            core_axis_name=("c", "s"),
            dimension_semantics=(pltpu.PARALLEL,),
        )(i_hbm)

        plsc.subcore_barrier()
        pltpu.sync_copy(acc.at[pl.ds(sid * rows, rows)],
                        o_hbm.at[cid].at[pl.ds(sid * rows, rows)])

    return k(idx, ones, zeros)


_BN = 1000
_BE = 4000


def _silu(x):
    return x * jax.nn.sigmoid(x)


def _onehot_from(ids, nclass):
    return (ids[:, None] == lax.broadcasted_iota(jnp.int32, (ids.shape[0], nclass), 1)
            ).astype(jnp.float32)


def _tc_tables(lat9, wlat_all, emb_pad, wtop, t, wbot):
    g = lat9.shape[0]

    def body(l_ref, wl_ref, e_ref, wt_ref, t_ref, wb_ref,
             latt_ref, t1_ref, t2_ref):
        l = l_ref[...]
        cols = []
        for i in range(3):
            for kk in range(3):
                acc = l[:, 3 * i + 0] * l[:, 3 * kk + 0]
                acc = acc + l[:, 3 * i + 1] * l[:, 3 * kk + 1]
                acc = acc + l[:, 3 * i + 2] * l[:, 3 * kk + 2]
                cols.append(acc)
        for _ in range(7):
            cols.append(jnp.zeros((g,), jnp.float32))
        lat_ip = jnp.stack(cols, axis=1)
        for layer in range(4):
            wl = wl_ref[pl.ds(16 * layer, 16), :]
            latt_ref[pl.ds(g * layer, g), :] = jnp.dot(
                lat_ip, wl, preferred_element_type=jnp.float32)
        t1_ref[...] = jnp.dot(e_ref[...], wt_ref[...],
                              preferred_element_type=jnp.float32)
        t2_ref[...] = jnp.dot(t_ref[...], wb_ref[...],
                              preferred_element_type=jnp.float32)

    return pl.pallas_call(
        body,
        out_shape=(jax.ShapeDtypeStruct((4 * g, 128), jnp.float32),
                   jax.ShapeDtypeStruct((128, 128), jnp.float32),
                   jax.ShapeDtypeStruct((g, 128), jnp.float32)),
    )(lat9, wlat_all, emb_pad, wtop, t, wbot)


def _tc_h0(at3, n2g3, t1, t2, b, latt0, wi, wj, n):
    nb = n // _BN

    def body(at_ref, ng_ref, t1_ref, t2_ref, b_ref, lt_ref, wi_ref, wj_ref,
             h_ref, hs_ref, hd_ref):
        oh_at = _onehot_from(at_ref[0, 0, :], 128)
        oh_g = _onehot_from(ng_ref[0, 0, :], 64)
        h = (jnp.dot(oh_at, t1_ref[...], preferred_element_type=jnp.float32)
             + jnp.dot(oh_g, t2_ref[...], preferred_element_type=jnp.float32)
             + b_ref[...])
        h_ref[...] = h
        hs_ref[...] = (jnp.dot(h, wi_ref[...], preferred_element_type=jnp.float32)
                       + jnp.dot(oh_g, lt_ref[...], preferred_element_type=jnp.float32))
        hd_ref[...] = jnp.dot(h, wj_ref[...], preferred_element_type=jnp.float32)

    fixed = pl.BlockSpec(None, None)
    return pl.pallas_call(
        body,
        grid=(nb,),
        in_specs=[pl.BlockSpec((1, 1, _BN), lambda i: (i, 0, 0)),
                  pl.BlockSpec((1, 1, _BN), lambda i: (i, 0, 0)),
                  fixed, fixed, fixed, fixed, fixed, fixed],
        out_specs=[pl.BlockSpec((_BN, 128), lambda i: (i, 0)),
                   pl.BlockSpec((_BN, 128), lambda i: (i, 0)),
                   pl.BlockSpec((_BN, 128), lambda i: (i, 0))],
        out_shape=(jax.ShapeDtypeStruct((n, 128), jnp.float32),
                   jax.ShapeDtypeStruct((n, 128), jnp.float32),
                   jax.ShapeDtypeStruct((n, 128), jnp.float32)),
    )(at3, n2g3, t1, t2, b, latt0, wi, wj)


def _tc_dis(fdT, fmapT):
    e = fdT.shape[1]
    be = 6400
    nb = e // be

    def body(d_ref, f_ref, o_ref):
        ang_t = jnp.dot(f_ref[...], d_ref[...],
                        preferred_element_type=jnp.float32)
        row = lax.broadcasted_iota(jnp.int32, ang_t.shape, 0)
        dis_t = jnp.where(row < 32, jnp.sin(ang_t), jnp.cos(ang_t))
        o_ref[...] = dis_t.T.astype(jnp.bfloat16)

    fixed = pl.BlockSpec(None, None)
    return pl.pallas_call(
        body,
        grid=(nb,),
        in_specs=[pl.BlockSpec((4, be), lambda i: (0, i)),
                  fixed],
        out_specs=pl.BlockSpec((be, 64), lambda i: (i, 0)),
        out_shape=jax.ShapeDtypeStruct((e, 64), jnp.bfloat16),
    )(fdT, fmapT)


def _tc_edge(gsd, dis, wdis, w2, b1, b2):
    e = gsd.shape[0]
    nb = e // _BE

    def body(s_ref, x_ref, wd_ref, w2_ref, b1_ref, b2_ref, o_ref):
        pre = (s_ref[...] + b1_ref[...]
               + jnp.dot(x_ref[...].astype(jnp.float32), wd_ref[...],
                         preferred_element_type=jnp.float32))
        s1 = _silu(pre)
        z = jnp.dot(s1, w2_ref[...], preferred_element_type=jnp.float32) + b2_ref[...]
        o_ref[...] = _silu(z)

    fixed = pl.BlockSpec(None, None)
    return pl.pallas_call(
        body,
        grid=(nb,),
        in_specs=[pl.BlockSpec((_BE, 128), lambda i: (i, 0)),
                  pl.BlockSpec((_BE, 64), lambda i: (i, 0)),
                  fixed, fixed, fixed, fixed],
        out_specs=pl.BlockSpec((_BE, 128), lambda i: (i, 0)),
        out_shape=jax.ShapeDtypeStruct((e, 128), jnp.float32),
    )(gsd, dis, wdis, w2, b1, b2)


def _tc_rdenom(cnt2):
    npd = cnt2.shape[1]
    bn = 632

    def body(c_ref, o_ref):
        c = c_ref[0, :, 0:1] + c_ref[1, :, 0:1]
        o_ref[...] = jnp.broadcast_to(1.0 / jnp.maximum(c, 1.0), (bn, 8))

    return pl.pallas_call(
        body,
        grid=(npd // bn,),
        in_specs=[pl.BlockSpec((2, bn, 128), lambda i: (0, i, 0))],
        out_specs=pl.BlockSpec((bn, 8), lambda i: (i, 0)),
        out_shape=jax.ShapeDtypeStruct((npd, 8), jnp.float32),
    )(cnt2)


def _tc_node(h, agg2, rd, w1h, w1a, b1, w2, b2, nxt):
    n = h.shape[0]
    nb = n // _BN

    def body(h_ref, a_ref, c_ref, w1h_ref, w1a_ref, b1_ref, w2_ref, b2_ref,
             *rest):
        h = h_ref[...]
        agg = (a_ref[0] + a_ref[1]) * c_ref[:, 0:1]
        z1 = _silu(jnp.dot(h, w1h_ref[...], preferred_element_type=jnp.float32)
                   + jnp.dot(agg, w1a_ref[...], preferred_element_type=jnp.float32)
                   + b1_ref[...])
        z2 = _silu(jnp.dot(z1, w2_ref[...], preferred_element_type=jnp.float32)
                   + b2_ref[...])
        hn = h + z2
        if nxt is None:
            (ho_ref,) = rest
            ho_ref[...] = hn
        else:
            ng_ref, lt_ref, wi_ref, wj_ref, ho_ref, hs_ref, hd_ref = rest
            ho_ref[...] = hn
            oh_g = _onehot_from(ng_ref[0, 0, :], 64)
            hs_ref[...] = (jnp.dot(hn, wi_ref[...], preferred_element_type=jnp.float32)
                           + jnp.dot(oh_g, lt_ref[...], preferred_element_type=jnp.float32))
            hd_ref[...] = jnp.dot(hn, wj_ref[...], preferred_element_type=jnp.float32)

    fixed = pl.BlockSpec(None, None)
    in_specs = [pl.BlockSpec((_BN, 128), lambda i: (i, 0)),
                pl.BlockSpec((2, _BN, 128), lambda i: (0, i, 0)),
                pl.BlockSpec((_BN, 8), lambda i: (i, 0)),
                fixed, fixed, fixed, fixed, fixed]
    args = [h, agg2, rd, w1h, w1a, b1, w2, b2]
    out_specs = [pl.BlockSpec((_BN, 128), lambda i: (i, 0))]
    out_shape = [jax.ShapeDtypeStruct((n, 128), jnp.float32)]
    if nxt is not None:
        n2g3, latt_l, wi, wj = nxt
        in_specs += [pl.BlockSpec((1, 1, _BN), lambda i: (i, 0, 0)),
                     fixed, fixed, fixed]
        args += [n2g3, latt_l, wi, wj]
        out_specs += [pl.BlockSpec((_BN, 128), lambda i: (i, 0)),
                      pl.BlockSpec((_BN, 128), lambda i: (i, 0))]
        out_shape += [jax.ShapeDtypeStruct((n, 128), jnp.float32),
                      jax.ShapeDtypeStruct((n, 128), jnp.float32)]
    res = pl.pallas_call(
        body,
        grid=(nb,),
        in_specs=in_specs,
        out_specs=out_specs,
        out_shape=out_shape,
    )(*args)
    return res if nxt is not None else (res[0],)


def _tc_pool(h, n2g3, coordw, n):
    nb = n // _BN

    def body(h_ref, ng_ref, cw_ref, co_ref, gs_ref, gc_ref):
        i = pl.program_id(0)
        h = h_ref[...]
        co_ref[...] = jnp.dot(h, cw_ref[...], preferred_element_type=jnp.float32)
        ids = ng_ref[0, 0, :]
        oht = (lax.broadcasted_iota(jnp.int32, (64, _BN), 0) == ids[None, :]
               ).astype(jnp.float32)

        @pl.when(i == 0)
        def _():
            gs_ref[...] = jnp.zeros_like(gs_ref)
            gc_ref[...] = jnp.zeros_like(gc_ref)

        gs_ref[...] += jnp.dot(oht, h, preferred_element_type=jnp.float32)
        gc_ref[...] += jnp.dot(oht, jnp.ones((_BN, 128), jnp.float32),
                               preferred_element_type=jnp.float32)

    fixed = pl.BlockSpec(None, None)
    return pl.pallas_call(
        body,
        grid=(nb,),
        in_specs=[pl.BlockSpec((_BN, 128), lambda i: (i, 0)),
                  pl.BlockSpec((1, 1, _BN), lambda i: (i, 0, 0)),
                  fixed],
        out_specs=[pl.BlockSpec((_BN, 8), lambda i: (i, 0)),
                   pl.BlockSpec((64, 128), lambda i: (0, 0)),
                   pl.BlockSpec((64, 128), lambda i: (0, 0))],
        out_shape=(jax.ShapeDtypeStruct((n, 8), jnp.float32),
                   jax.ShapeDtypeStruct((64, 128), jnp.float32),
                   jax.ShapeDtypeStruct((64, 128), jnp.float32)),
    )(h, n2g3, coordw)


def _tc_lattice(gsum, gcnt, latw, lat9):
    g = lat9.shape[0]

    def body(gs_ref, gc_ref, w_ref, l_ref, o_ref):
        gf = gs_ref[...] / jnp.maximum(gc_ref[...], 1.0)
        gl = jnp.dot(gf, w_ref[...], preferred_element_type=jnp.float32)
        l = l_ref[...]
        cols = []
        for i in range(3):
            for kk in range(3):
                acc = gl[:, 3 * i + 0] * l[:, 0 + kk]
                acc = acc + gl[:, 3 * i + 1] * l[:, 3 + kk]
                acc = acc + gl[:, 3 * i + 2] * l[:, 6 + kk]
                cols.append(acc)
        for _ in range(7):
            cols.append(jnp.zeros((g,), jnp.float32))
        o_ref[...] = jnp.stack(cols, axis=1)

    return pl.pallas_call(
        body,
        out_shape=jax.ShapeDtypeStruct((g, 16), jnp.float32),
    )(gsum, gcnt, latw, lat9)


def kernel(atom_types, frac_coords, lattices, edge_index, node2graph, t, params):
    n = atom_types.shape[0]
    e = edge_index.shape[1]
    g = lattices.shape[0]
    hid = 128

    src = edge_index[0].astype(jnp.int32).reshape(1, e)
    dst = edge_index[1].astype(jnp.int32).reshape(1, e)
    at3 = atom_types.astype(jnp.int32).reshape(n // _BN, 1, _BN)
    n2g3 = node2graph.astype(jnp.int32).reshape(n // _BN, 1, _BN)

    p = params
    layers = p["layers"]

    wtop = p["atom_latent_W"][:hid]
    wbot = p["atom_latent_W"][hid:]
    b_al = p["atom_latent_b"].reshape(1, hid)
    emb_pad = jnp.zeros((128, hid), jnp.float32).at[:p["node_emb"].shape[0]].set(
        p["node_emb"])
    lat9 = jnp.pad(lattices.reshape(g, 9), ((0, 0), (0, 7)))
    wlat_all = jnp.concatenate(
        [jnp.pad(lp["eW1"][2 * hid:2 * hid + 9], ((0, 7), (0, 0)))
         for lp in layers], axis=0)
    wi = [lp["eW1"][:hid] for lp in layers]
    wj = [lp["eW1"][hid:2 * hid] for lp in layers]
    wdis = []
    for lp in layers:
        wd = lp["eW1"][2 * hid + 9:]
        wdis.append(jnp.concatenate([
            wd[:30], jnp.zeros((2, hid), jnp.float32),
            wd[30:], jnp.zeros((2, hid), jnp.float32)], axis=0))
    eb1 = [lp["eb1"].reshape(1, hid) for lp in layers]
    ew2 = [lp["eW2"] for lp in layers]
    eb2 = [lp["eb2"].reshape(1, hid) for lp in layers]
    nw1h = [lp["nW1"][:hid] for lp in layers]
    nw1a = [lp["nW1"][hid:] for lp in layers]
    nb1 = [lp["nb1"].reshape(1, hid) for lp in layers]
    nw2 = [lp["nW2"] for lp in layers]
    nb2 = [lp["nb2"].reshape(1, hid) for lp in layers]
    coordw = jnp.pad(p["coord_W"], ((0, 0), (0, 5)))
    latw = jnp.pad(p["lattice_W"], ((0, 0), (0, 7)))

    freqs = 2.0 * math.pi * np.arange(NFREQ, dtype=np.float32)
    fmap_np = np.zeros((64, 4), np.float32)
    for j in range(3):
        for f in range(NFREQ):
            fmap_np[j * NFREQ + f, j] = freqs[f]
            fmap_np[32 + j * NFREQ + f, j] = freqs[f]
    fmap_t = jnp.asarray(fmap_np)

    frac_flat = jnp.pad(frac_coords, ((0, 0), (0, 1))).reshape(-1)
    npad = ((n + 8 * _SC_TILES - 1) // (8 * _SC_TILES)) * (8 * _SC_TILES)
    zeros_2nd = jnp.zeros((_SC_CORES, npad, hid), jnp.float32)
    zeros_nc = jnp.zeros((npad, hid), jnp.float32)
    ones_w = jnp.ones((_SC_W, hid), jnp.float32)

    eh = e // 2
    src1 = lax.slice(src, (0, 0), (1, eh))
    src2 = lax.slice(src, (0, eh), (1, e))
    dst1 = lax.slice(dst, (0, 0), (1, eh))
    dst2 = lax.slice(dst, (0, eh), (1, e))

    latt_all, t1, t2 = _tc_tables(lat9, wlat_all, emb_pad, wtop, t, wbot)
    cnt2 = _sc_counts(src, ones_w, zeros_nc)
    rd = _tc_rdenom(cnt2)
    fdT1 = _sc_fdiff(frac_flat, src1, dst1)
    fdT2 = _sc_fdiff(frac_flat, src2, dst2)
    dis1 = _tc_dis(fdT1, fmap_t)
    dis2 = _tc_dis(fdT2, fmap_t)
    h, hs, hd = _tc_h0(at3, n2g3, t1, t2, b_al,
                       lax.slice_in_dim(latt_all, 0, g), wi[0], wj[0], n)

    for l in range(4):
        gsd1 = _sc_gather_add(hs, src1, hd, dst1)
        ef1 = _tc_edge(gsd1, dis1, wdis[l], ew2[l], eb1[l], eb2[l])
        gsd2 = _sc_gather_add(hs, src2, hd, dst2)
        agg2a = _sc_scatter_rows(ef1, src1, zeros_2nd)
        ef2 = _tc_edge(gsd2, dis2, wdis[l], ew2[l], eb1[l], eb2[l])
        agg2 = _sc_scatter_rows(ef2, src2, agg2a)
        if l < 3:
            nxt = (n2g3, lax.slice_in_dim(latt_all, (l + 1) * g, (l + 2) * g),
                   wi[l + 1], wj[l + 1])
            h, hs, hd = _tc_node(h, agg2, rd, nw1h[l], nw1a[l], nb1[l],
                                 nw2[l], nb2[l], nxt)
        else:
            (h,) = _tc_node(h, agg2, rd, nw1h[l], nw1a[l], nb1[l],
                            nw2[l], nb2[l], None)

    coord8, gsum, gcnt = _tc_pool(h, n2g3, coordw, n)
    lo16 = _tc_lattice(gsum, gcnt, latw, lat9)

    coord_out = coord8[:, :3]
    lattice_out = lo16[:, :9].reshape(g, 3, 3)
    return lattice_out, coord_out


# speedup vs baseline: 13.6419x; 1.0007x over previous
import dataclasses
import functools
import math

import jax
import jax.numpy as jnp
import numpy as np
from jax import lax
from jax.experimental import pallas as pl
from jax.experimental.pallas import tpu as pltpu
from jax.experimental.pallas import tpu_sc as plsc

NFREQ = 10


_SC_CORES = 2
_SC_TILES = 16
_SC_W = 128


def _sc_mesh():
    return plsc.VectorSubcoreMesh(
        core_axis_name="c", subcore_axis_name="s",
        num_cores=_SC_CORES, num_subcores=_SC_TILES)


def _sc_gather_add(t1, i1, t2, i2):
    n, d = t1.shape
    e = i1.shape[1]
    w = _SC_W

    @functools.partial(
        pl.kernel,
        out_type=jax.ShapeDtypeStruct((e, d), jnp.float32),
        mesh=_sc_mesh(),
        scratch_types=[pltpu.VMEM_SHARED((n, d), jnp.float32)])
    def k(t1_hbm, i1_hbm, t2_hbm, i2_hbm, o_hbm, t1s):
        sid = lax.axis_index("s")

        @pl.when(sid == 0)
        def _():
            pltpu.sync_copy(t1_hbm, t1s)

        plsc.subcore_barrier()

        def body(i1_v, i2_v, o_v):
            pltpu.sync_copy(t1s.at[i1_v.at[0]], o_v)
            pltpu.sync_copy(t2_hbm.at[i2_v.at[0]], o_v, add=True)

        pltpu.emit_pipeline(
            body,
            grid=(e // w,),
            in_specs=[pl.BlockSpec((1, w), lambda i: (0, i)),
                      pl.BlockSpec((1, w), lambda i: (0, i))],
            out_specs=[pl.BlockSpec((w, d), lambda i: (i, 0))],
            core_axis_name=("c", "s"),
            dimension_semantics=(pltpu.PARALLEL,),
        )(i1_hbm, i2_hbm, o_hbm)

    return k(t1, i1, t2, i2)


def _sc_fdiff(frac_flat, src, dst):
    e = src.shape[1]
    w = _SC_W
    nflat = frac_flat.shape[0]

    cp = pltpu.CompilerParams()
    if "needs_layout_passes" in pltpu.CompilerParams.__dataclass_fields__:
        cp = dataclasses.replace(cp, needs_layout_passes=False)

    @functools.partial(
        pl.kernel,
        out_type=jax.ShapeDtypeStruct((4, e), jnp.float32),
        mesh=_sc_mesh(),
        compiler_params=cp,
        scratch_types=[pltpu.VMEM((nflat,), jnp.float32)])
    def k(f_hbm, s_hbm, d_hbm, o_hbm, tbl):
        pltpu.sync_copy(f_hbm, tbl)

        def body(s_v, d_v, o_v):
            for gi in range(w // 16):
                sl = pl.ds(gi * 16, 16)
                s16 = s_v[0, sl] * 4
                d16 = d_v[0, sl] * 4
                for c in range(3):
                    fs = plsc.load_gather(tbl, [s16 + c])
                    fd = plsc.load_gather(tbl, [d16 + c])
                    o_v[c, sl] = fd - fs
                o_v[3, sl] = jnp.zeros((16,), jnp.float32)

        pltpu.emit_pipeline(
            body,
            grid=(e // w,),
            in_specs=[pl.BlockSpec((1, w), lambda i: (0, i)),
                      pl.BlockSpec((1, w), lambda i: (0, i))],
            out_specs=[pl.BlockSpec((4, w), lambda i: (0, i))],
            core_axis_name=("c", "s"),
            dimension_semantics=(pltpu.PARALLEL,),
        )(s_hbm, d_hbm, o_hbm)

    return k(frac_flat, src, dst)


def _sc_scatter_rows(vals, idx, init):
    e, d = vals.shape
    n = init.shape[1]
    w = _SC_W
    rows = n // _SC_TILES

    @functools.partial(
        pl.kernel,
        out_type=jax.ShapeDtypeStruct((_SC_CORES, n, d), jnp.float32),
        mesh=_sc_mesh(),
        scratch_types=[pltpu.VMEM_SHARED((n, d), jnp.float32)])
    def k(v_hbm, i_hbm, z_hbm, o_hbm, acc):
        cid = lax.axis_index("c")
        sid = lax.axis_index("s")
        pltpu.sync_copy(z_hbm.at[cid].at[pl.ds(sid * rows, rows)],
                        acc.at[pl.ds(sid * rows, rows)])
        plsc.subcore_barrier()

        def body(v_v, i_v):
            pltpu.sync_copy(v_v, acc.at[i_v.at[0]], add=True)

        pltpu.emit_pipeline(
            body,
            grid=(e // w,),
            in_specs=[pl.BlockSpec((w, d), lambda i: (i, 0)),
                      pl.BlockSpec((1, w), lambda i: (0, i))],
            out_specs=[],
            core_axis_name=("c", "s"),
            dimension_semantics=(pltpu.PARALLEL,),
        )(v_hbm, i_hbm)

        plsc.subcore_barrier()
        pltpu.sync_copy(acc.at[pl.ds(sid * rows, rows)],
                        o_hbm.at[cid].at[pl.ds(sid * rows, rows)])

    return k(vals, idx, init)


def _sc_counts(idx, ones, zeros):
    e = idx.shape[1]
    n, dc = zeros.shape
    w = _SC_W
    rows = n // _SC_TILES

    @functools.partial(
        pl.kernel,
        out_type=jax.ShapeDtypeStruct((_SC_CORES, n, dc), jnp.float32),
        mesh=_sc_mesh(),
        scratch_types=[pltpu.VMEM((w, dc), jnp.float32),
                       pltpu.VMEM_SHARED((n, dc), jnp.float32)])
    def k(i_hbm, one_hbm, z_hbm, o_hbm, ones_v, acc):
        cid = lax.axis_index("c")
        sid = lax.axis_index("s")
        pltpu.sync_copy(one_hbm, ones_v)
        pltpu.sync_copy(z_hbm.at[pl.ds(sid * rows, rows)],
                        acc.at[pl.ds(sid * rows, rows)])
        plsc.subcore_barrier()

        def body(i_v):
            pltpu.sync_copy(ones_v, acc.at[i_v.at[0]], add=True)

        pltpu.emit_pipeline(
            body,
            grid=(e // w,),
            in_specs=[pl.BlockSpec((1, w), lambda i: (0, i))],
            out_specs=[],
            core_axis_name=("c", "s"),
            dimension_semantics=(pltpu.PARALLEL,),
        )(i_hbm)

        plsc.subcore_barrier()
        pltpu.sync_copy(acc.at[pl.ds(sid * rows, rows)],
                        o_hbm.at[cid].at[pl.ds(sid * rows, rows)])

    return k(idx, ones, zeros)


_BN = 1000
_BE = 4000


def _silu(x):
    return x * jax.nn.sigmoid(x)


def _onehot_from(ids, nclass):
    return (ids[:, None] == lax.broadcasted_iota(jnp.int32, (ids.shape[0], nclass), 1)
            ).astype(jnp.float32)


def _tc_tables(lat9, wlat_all, emb_pad, wtop, t, wbot):
    g = lat9.shape[0]

    def body(l_ref, wl_ref, e_ref, wt_ref, t_ref, wb_ref,
             latt_ref, t1_ref, t2_ref):
        l = l_ref[...]
        cols = []
        for i in range(3):
            for kk in range(3):
                acc = l[:, 3 * i + 0] * l[:, 3 * kk + 0]
                acc = acc + l[:, 3 * i + 1] * l[:, 3 * kk + 1]
                acc = acc + l[:, 3 * i + 2] * l[:, 3 * kk + 2]
                cols.append(acc)
        for _ in range(7):
            cols.append(jnp.zeros((g,), jnp.float32))
        lat_ip = jnp.stack(cols, axis=1)
        for layer in range(4):
            wl = wl_ref[pl.ds(16 * layer, 16), :]
            latt_ref[pl.ds(g * layer, g), :] = jnp.dot(
                lat_ip, wl, preferred_element_type=jnp.float32)
        t1_ref[...] = jnp.dot(e_ref[...], wt_ref[...],
                              preferred_element_type=jnp.float32)
        t2_ref[...] = jnp.dot(t_ref[...], wb_ref[...],
                              preferred_element_type=jnp.float32)

    return pl.pallas_call(
        body,
        out_shape=(jax.ShapeDtypeStruct((4 * g, 128), jnp.float32),
                   jax.ShapeDtypeStruct((128, 128), jnp.float32),
                   jax.ShapeDtypeStruct((g, 128), jnp.float32)),
    )(lat9, wlat_all, emb_pad, wtop, t, wbot)


def _tc_h0(at3, n2g3, t1, t2, b, latt0, wi, wj, n):
    nb = n // _BN

    def body(at_ref, ng_ref, t1_ref, t2_ref, b_ref, lt_ref, wi_ref, wj_ref,
             h_ref, hs_ref, hd_ref):
        oh_at = _onehot_from(at_ref[0, 0, :], 128)
        oh_g = _onehot_from(ng_ref[0, 0, :], 64)
        h = (jnp.dot(oh_at, t1_ref[...], preferred_element_type=jnp.float32)
             + jnp.dot(oh_g, t2_ref[...], preferred_element_type=jnp.float32)
             + b_ref[...])
        h_ref[...] = h
        hs_ref[...] = (jnp.dot(h, wi_ref[...], preferred_element_type=jnp.float32)
                       + jnp.dot(oh_g, lt_ref[...], preferred_element_type=jnp.float32))
        hd_ref[...] = jnp.dot(h, wj_ref[...], preferred_element_type=jnp.float32)

    fixed = pl.BlockSpec(None, None)
    return pl.pallas_call(
        body,
        grid=(nb,),
        in_specs=[pl.BlockSpec((1, 1, _BN), lambda i: (i, 0, 0)),
                  pl.BlockSpec((1, 1, _BN), lambda i: (i, 0, 0)),
                  fixed, fixed, fixed, fixed, fixed, fixed],
        out_specs=[pl.BlockSpec((_BN, 128), lambda i: (i, 0)),
                   pl.BlockSpec((_BN, 128), lambda i: (i, 0)),
                   pl.BlockSpec((_BN, 128), lambda i: (i, 0))],
        out_shape=(jax.ShapeDtypeStruct((n, 128), jnp.float32),
                   jax.ShapeDtypeStruct((n, 128), jnp.float32),
                   jax.ShapeDtypeStruct((n, 128), jnp.float32)),
    )(at3, n2g3, t1, t2, b, latt0, wi, wj)


def _tc_dis(fdT, fmapT):
    e = fdT.shape[1]
    be = 6400
    nb = e // be

    def body(d_ref, f_ref, o_ref):
        ang_t = jnp.dot(f_ref[...], d_ref[...],
                        preferred_element_type=jnp.float32)
        row = lax.broadcasted_iota(jnp.int32, ang_t.shape, 0)
        dis_t = jnp.where(row < 32, jnp.sin(ang_t), jnp.cos(ang_t))
        o_ref[...] = dis_t.T.astype(jnp.bfloat16)

    fixed = pl.BlockSpec(None, None)
    return pl.pallas_call(
        body,
        grid=(nb,),
        in_specs=[pl.BlockSpec((4, be), lambda i: (0, i)),
                  fixed],
        out_specs=pl.BlockSpec((be, 64), lambda i: (i, 0)),
        out_shape=jax.ShapeDtypeStruct((e, 64), jnp.bfloat16),
    )(fdT, fmapT)


def _tc_edge(gsd, dis, wdis, w2, b1, b2):
    e = gsd.shape[0]
    nb = e // _BE

    def body(s_ref, x_ref, wd_ref, w2_ref, b1_ref, b2_ref, o_ref):
        pre = (s_ref[...] + b1_ref[...]
               + jnp.dot(x_ref[...].astype(jnp.float32), wd_ref[...],
                         preferred_element_type=jnp.float32))
        s1 = _silu(pre)
        z = jnp.dot(s1, w2_ref[...], preferred_element_type=jnp.float32) + b2_ref[...]
        o_ref[...] = _silu(z)

    fixed = pl.BlockSpec(None, None)
    return pl.pallas_call(
        body,
        grid=(nb,),
        in_specs=[pl.BlockSpec((_BE, 128), lambda i: (i, 0)),
                  pl.BlockSpec((_BE, 64), lambda i: (i, 0)),
                  fixed, fixed, fixed, fixed],
        out_specs=pl.BlockSpec((_BE, 128), lambda i: (i, 0)),
        out_shape=jax.ShapeDtypeStruct((e, 128), jnp.float32),
    )(gsd, dis, wdis, w2, b1, b2)


def _tc_rdenom(cnt2):
    npd = cnt2.shape[1]
    bn = 632

    dc = cnt2.shape[2]

    def body(c_ref, o_ref):
        c = c_ref[0, :, 0:1] + c_ref[1, :, 0:1]
        o_ref[...] = jnp.broadcast_to(1.0 / jnp.maximum(c, 1.0), (bn, 8))

    return pl.pallas_call(
        body,
        grid=(npd // bn,),
        in_specs=[pl.BlockSpec((2, bn, dc), lambda i: (0, i, 0))],
        out_specs=pl.BlockSpec((bn, 8), lambda i: (i, 0)),
        out_shape=jax.ShapeDtypeStruct((npd, 8), jnp.float32),
    )(cnt2)


def _tc_node(h, agg2, rd, w1h, w1a, b1, w2, b2, nxt):
    n = h.shape[0]
    nb = n // _BN

    def body(h_ref, a_ref, c_ref, w1h_ref, w1a_ref, b1_ref, w2_ref, b2_ref,
             *rest):
        h = h_ref[...]
        agg = (a_ref[0] + a_ref[1]) * c_ref[:, 0:1]
        z1 = _silu(jnp.dot(h, w1h_ref[...], preferred_element_type=jnp.float32)
                   + jnp.dot(agg, w1a_ref[...], preferred_element_type=jnp.float32)
                   + b1_ref[...])
        z2 = _silu(jnp.dot(z1, w2_ref[...], preferred_element_type=jnp.float32)
                   + b2_ref[...])
        hn = h + z2
        if nxt is None:
            (ho_ref,) = rest
            ho_ref[...] = hn
        else:
            ng_ref, lt_ref, wi_ref, wj_ref, ho_ref, hs_ref, hd_ref = rest
            ho_ref[...] = hn
            oh_g = _onehot_from(ng_ref[0, 0, :], 64)
            hs_ref[...] = (jnp.dot(hn, wi_ref[...], preferred_element_type=jnp.float32)
                           + jnp.dot(oh_g, lt_ref[...], preferred_element_type=jnp.float32))
            hd_ref[...] = jnp.dot(hn, wj_ref[...], preferred_element_type=jnp.float32)

    fixed = pl.BlockSpec(None, None)
    in_specs = [pl.BlockSpec((_BN, 128), lambda i: (i, 0)),
                pl.BlockSpec((2, _BN, 128), lambda i: (0, i, 0)),
                pl.BlockSpec((_BN, 8), lambda i: (i, 0)),
                fixed, fixed, fixed, fixed, fixed]
    args = [h, agg2, rd, w1h, w1a, b1, w2, b2]
    out_specs = [pl.BlockSpec((_BN, 128), lambda i: (i, 0))]
    out_shape = [jax.ShapeDtypeStruct((n, 128), jnp.float32)]
    if nxt is not None:
        n2g3, latt_l, wi, wj = nxt
        in_specs += [pl.BlockSpec((1, 1, _BN), lambda i: (i, 0, 0)),
                     fixed, fixed, fixed]
        args += [n2g3, latt_l, wi, wj]
        out_specs += [pl.BlockSpec((_BN, 128), lambda i: (i, 0)),
                      pl.BlockSpec((_BN, 128), lambda i: (i, 0))]
        out_shape += [jax.ShapeDtypeStruct((n, 128), jnp.float32),
                      jax.ShapeDtypeStruct((n, 128), jnp.float32)]
    res = pl.pallas_call(
        body,
        grid=(nb,),
        in_specs=in_specs,
        out_specs=out_specs,
        out_shape=out_shape,
    )(*args)
    return res if nxt is not None else (res[0],)


def _tc_pool(h, n2g3, coordw, n):
    nb = n // _BN

    def body(h_ref, ng_ref, cw_ref, co_ref, gs_ref, gc_ref):
        i = pl.program_id(0)
        h = h_ref[...]
        co_ref[...] = jnp.dot(h, cw_ref[...], preferred_element_type=jnp.float32)
        ids = ng_ref[0, 0, :]
        oht = (lax.broadcasted_iota(jnp.int32, (64, _BN), 0) == ids[None, :]
               ).astype(jnp.float32)

        @pl.when(i == 0)
        def _():
            gs_ref[...] = jnp.zeros_like(gs_ref)
            gc_ref[...] = jnp.zeros_like(gc_ref)

        gs_ref[...] += jnp.dot(oht, h, preferred_element_type=jnp.float32)
        gc_ref[...] += jnp.dot(oht, jnp.ones((_BN, 128), jnp.float32),
                               preferred_element_type=jnp.float32)

    fixed = pl.BlockSpec(None, None)
    return pl.pallas_call(
        body,
        grid=(nb,),
        in_specs=[pl.BlockSpec((_BN, 128), lambda i: (i, 0)),
                  pl.BlockSpec((1, 1, _BN), lambda i: (i, 0, 0)),
                  fixed],
        out_specs=[pl.BlockSpec((_BN, 8), lambda i: (i, 0)),
                   pl.BlockSpec((64, 128), lambda i: (0, 0)),
                   pl.BlockSpec((64, 128), lambda i: (0, 0))],
        out_shape=(jax.ShapeDtypeStruct((n, 8), jnp.float32),
                   jax.ShapeDtypeStruct((64, 128), jnp.float32),
                   jax.ShapeDtypeStruct((64, 128), jnp.float32)),
    )(h, n2g3, coordw)


def _tc_lattice(gsum, gcnt, latw, lat9):
    g = lat9.shape[0]

    def body(gs_ref, gc_ref, w_ref, l_ref, o_ref):
        gf = gs_ref[...] / jnp.maximum(gc_ref[...], 1.0)
        gl = jnp.dot(gf, w_ref[...], preferred_element_type=jnp.float32)
        l = l_ref[...]
        cols = []
        for i in range(3):
            for kk in range(3):
                acc = gl[:, 3 * i + 0] * l[:, 0 + kk]
                acc = acc + gl[:, 3 * i + 1] * l[:, 3 + kk]
                acc = acc + gl[:, 3 * i + 2] * l[:, 6 + kk]
                cols.append(acc)
        for _ in range(7):
            cols.append(jnp.zeros((g,), jnp.float32))
        o_ref[...] = jnp.stack(cols, axis=1)

    return pl.pallas_call(
        body,
        out_shape=jax.ShapeDtypeStruct((g, 16), jnp.float32),
    )(gsum, gcnt, latw, lat9)


def kernel(atom_types, frac_coords, lattices, edge_index, node2graph, t, params):
    n = atom_types.shape[0]
    e = edge_index.shape[1]
    g = lattices.shape[0]
    hid = 128

    src = edge_index[0].astype(jnp.int32).reshape(1, e)
    dst = edge_index[1].astype(jnp.int32).reshape(1, e)
    at3 = atom_types.astype(jnp.int32).reshape(n // _BN, 1, _BN)
    n2g3 = node2graph.astype(jnp.int32).reshape(n // _BN, 1, _BN)

    p = params
    layers = p["layers"]

    wtop = p["atom_latent_W"][:hid]
    wbot = p["atom_latent_W"][hid:]
    b_al = p["atom_latent_b"].reshape(1, hid)
    emb_pad = jnp.zeros((128, hid), jnp.float32).at[:p["node_emb"].shape[0]].set(
        p["node_emb"])
    lat9 = jnp.pad(lattices.reshape(g, 9), ((0, 0), (0, 7)))
    wlat_all = jnp.concatenate(
        [jnp.pad(lp["eW1"][2 * hid:2 * hid + 9], ((0, 7), (0, 0)))
         for lp in layers], axis=0)
    wi = [lp["eW1"][:hid] for lp in layers]
    wj = [lp["eW1"][hid:2 * hid] for lp in layers]
    wdis = []
    for lp in layers:
        wd = lp["eW1"][2 * hid + 9:]
        wdis.append(jnp.concatenate([
            wd[:30], jnp.zeros((2, hid), jnp.float32),
            wd[30:], jnp.zeros((2, hid), jnp.float32)], axis=0))
    eb1 = [lp["eb1"].reshape(1, hid) for lp in layers]
    ew2 = [lp["eW2"] for lp in layers]
    eb2 = [lp["eb2"].reshape(1, hid) for lp in layers]
    nw1h = [lp["nW1"][:hid] for lp in layers]
    nw1a = [lp["nW1"][hid:] for lp in layers]
    nb1 = [lp["nb1"].reshape(1, hid) for lp in layers]
    nw2 = [lp["nW2"] for lp in layers]
    nb2 = [lp["nb2"].reshape(1, hid) for lp in layers]
    coordw = jnp.pad(p["coord_W"], ((0, 0), (0, 5)))
    latw = jnp.pad(p["lattice_W"], ((0, 0), (0, 7)))

    freqs = 2.0 * math.pi * np.arange(NFREQ, dtype=np.float32)
    fmap_np = np.zeros((64, 4), np.float32)
    for j in range(3):
        for f in range(NFREQ):
            fmap_np[j * NFREQ + f, j] = freqs[f]
            fmap_np[32 + j * NFREQ + f, j] = freqs[f]
    fmap_t = jnp.asarray(fmap_np)

    frac_flat = jnp.pad(frac_coords, ((0, 0), (0, 1))).reshape(-1)
    npad = ((n + 8 * _SC_TILES - 1) // (8 * _SC_TILES)) * (8 * _SC_TILES)
    zeros_2nd = jnp.zeros((_SC_CORES, npad, hid), jnp.float32)
    zeros_nc = jnp.zeros((npad, hid), jnp.float32)
    ones_w = jnp.ones((_SC_W, hid), jnp.float32)

    eh = e // 2
    src1 = lax.slice(src, (0, 0), (1, eh))
    src2 = lax.slice(src, (0, eh), (1, e))
    dst1 = lax.slice(dst, (0, 0), (1, eh))
    dst2 = lax.slice(dst, (0, eh), (1, e))

    latt_all, t1, t2 = _tc_tables(lat9, wlat_all, emb_pad, wtop, t, wbot)
    cnt2 = _sc_counts(src, ones_w, zeros_nc)
    rd = _tc_rdenom(cnt2)
    fdT1 = _sc_fdiff(frac_flat, src1, dst1)
    fdT2 = _sc_fdiff(frac_flat, src2, dst2)
    dis1 = _tc_dis(fdT1, fmap_t)
    dis2 = _tc_dis(fdT2, fmap_t)
    h, hs, hd = _tc_h0(at3, n2g3, t1, t2, b_al,
                       lax.slice_in_dim(latt_all, 0, g), wi[0], wj[0], n)

    for l in range(4):
        gsd1 = _sc_gather_add(hs, src1, hd, dst1)
        ef1 = _tc_edge(gsd1, dis1, wdis[l], ew2[l], eb1[l], eb2[l])
        gsd2 = _sc_gather_add(hs, src2, hd, dst2)
        agg2a = _sc_scatter_rows(ef1, src1, zeros_2nd)
        ef2 = _tc_edge(gsd2, dis2, wdis[l], ew2[l], eb1[l], eb2[l])
        agg2 = _sc_scatter_rows(ef2, src2, agg2a)
        if l < 3:
            nxt = (n2g3, lax.slice_in_dim(latt_all, (l + 1) * g, (l + 2) * g),
                   wi[l + 1], wj[l + 1])
            h, hs, hd = _tc_node(h, agg2, rd, nw1h[l], nw1a[l], nb1[l],
                                 nw2[l], nb2[l], nxt)
        else:
            (h,) = _tc_node(h, agg2, rd, nw1h[l], nw1a[l], nb1[l],
                            nw2[l], nb2[l], None)

    coord8, gsum, gcnt = _tc_pool(h, n2g3, coordw, n)
    lo16 = _tc_lattice(gsum, gcnt, latw, lat9)

    coord_out = coord8[:, :3]
    lattice_out = lo16[:, :9].reshape(g, 3, 3)
    return lattice_out, coord_out
